# baseline (speedup 1.0000x reference)
import functools

import jax
import jax.numpy as jnp
from jax import lax
from jax.experimental import pallas as pl
from jax.experimental.pallas import tpu as pltpu
from jax.experimental.pallas import tpu_sc as plsc

VOCAB = 32000
D_A = 768
D_B = 768
N_HEADS = 12
D_H = 64
D_FF = 3072
N_POOL = 8192
R = 2
TOP_K = 8
D_K = 64
N_ASPECTS = 4
T = 2048
D_POOL = R * (D_A + D_B)

TT = 256
NT = T // TT
PT = 1024
NPT = N_POOL // PT
VT = 640
NVT = VOCAB // VT

_F32 = jnp.float32


def _ln_in(x, s, b):
    m = jnp.mean(x, axis=-1, keepdims=True)
    v = jnp.mean((x - m) ** 2, axis=-1, keepdims=True)
    return (x - m) * lax.rsqrt(v + 1e-5) * s + b


def _pos_enc_const(seq_len, d_model):
    pos = jnp.arange(seq_len)[:, None]
    i = jnp.arange(d_model // 2)[None, :]
    angle = pos / 10000 ** (2 * i / d_model)
    enc = jnp.concatenate([jnp.sin(angle), jnp.cos(angle)], axis=-1)
    return enc[:, :d_model].astype(_F32)


def _embed_gather(table, idx):
    info = plsc.get_sparse_core_info()
    nc, ns = info.num_cores, info.num_subcores
    nw = nc * ns
    bpw = T // nw
    mesh = plsc.VectorSubcoreMesh(core_axis_name="c", subcore_axis_name="s")

    @functools.partial(
        pl.kernel, mesh=mesh,
        out_type=jax.ShapeDtypeStruct((T, D_A), _F32),
        scratch_types=[
            pltpu.VMEM((bpw,), jnp.int32),
            pltpu.VMEM((bpw, D_A), _F32),
            pltpu.SemaphoreType.DMA,
        ],
    )
    def k(table_hbm, idx_hbm, out_hbm, idx_v, rows_v, sem):
        wid = lax.axis_index("s") * nc + lax.axis_index("c")
        base = wid * bpw
        pltpu.sync_copy(idx_hbm.at[pl.ds(base, bpw)], idx_v)
        pltpu.async_copy(table_hbm.at[idx_v], rows_v, sem).wait()
        pltpu.sync_copy(rows_v, out_hbm.at[pl.ds(base, bpw)])

    return k(table, idx)


def _pool_gather(pool, idx):
    info = plsc.get_sparse_core_info()
    nc = info.num_cores
    mesh = plsc.VectorSubcoreMesh(core_axis_name="c", subcore_axis_name="s")

    @functools.partial(
        pl.kernel, mesh=mesh,
        out_type=jax.ShapeDtypeStruct((TOP_K, D_POOL), _F32),
        scratch_types=[
            pltpu.VMEM((TOP_K,), jnp.int32),
            pltpu.VMEM((TOP_K, D_POOL), _F32),
            pltpu.SemaphoreType.DMA,
        ],
    )
    def k(pool_hbm, idx_hbm, out_hbm, idx_v, rows_v, sem):
        wid = lax.axis_index("s") * nc + lax.axis_index("c")

        @pl.when(wid == 0)
        def _():
            pltpu.sync_copy(idx_hbm, idx_v)
            pltpu.async_copy(pool_hbm.at[idx_v], rows_v, sem).wait()
            pltpu.sync_copy(rows_v, out_hbm)

    return k(pool, idx)


def _qkv_a_call(g, pos, s1, b1, wqkv):
    def body(g_ref, p_ref, s_ref, b_ref, w_ref, q_ref, k_ref, v_ref, x_ref):
        x = g_ref[...] + p_ref[...]
        x_ref[...] = x
        h = _ln_in(x, s_ref[...], b_ref[...])
        qkv = jnp.dot(h, w_ref[...], preferred_element_type=_F32)
        q_ref[...] = qkv[:, :D_A]
        k_ref[...] = qkv[:, D_A:2 * D_A]
        v_ref[...] = qkv[:, 2 * D_A:]

    tile = lambda i: (i, 0)
    full = lambda i: (0, 0)
    return pl.pallas_call(
        body,
        grid=(NT,),
        in_specs=[
            pl.BlockSpec((TT, D_A), tile),
            pl.BlockSpec((TT, D_A), tile),
            pl.BlockSpec((1, D_A), full),
            pl.BlockSpec((1, D_A), full),
            pl.BlockSpec((D_A, 3 * D_A), full),
        ],
        out_specs=[
            pl.BlockSpec((TT, D_A), tile),
            pl.BlockSpec((TT, D_A), tile),
            pl.BlockSpec((TT, D_A), tile),
            pl.BlockSpec((TT, D_A), tile),
        ],
        out_shape=[jax.ShapeDtypeStruct((T, D_A), _F32)] * 4,
    )(g, pos, s1, b1, wqkv)


def _qkv_b_call(x, s1, b1, wqkv):
    def body(x_ref, s_ref, b_ref, w_ref, q_ref, k_ref, v_ref):
        h = _ln_in(x_ref[...], s_ref[...], b_ref[...])
        qkv = jnp.dot(h, w_ref[...], preferred_element_type=_F32)
        q_ref[...] = qkv[:, :D_A]
        k_ref[...] = qkv[:, D_A:2 * D_A]
        v_ref[...] = qkv[:, 2 * D_A:]

    tile = lambda i: (i, 0)
    full = lambda i: (0, 0)
    return pl.pallas_call(
        body,
        grid=(NT,),
        in_specs=[
            pl.BlockSpec((TT, D_A), tile),
            pl.BlockSpec((1, D_A), full),
            pl.BlockSpec((1, D_A), full),
            pl.BlockSpec((D_A, 3 * D_A), full),
        ],
        out_specs=[
            pl.BlockSpec((TT, D_A), tile),
            pl.BlockSpec((TT, D_A), tile),
            pl.BlockSpec((TT, D_A), tile),
        ],
        out_shape=[jax.ShapeDtypeStruct((T, D_A), _F32)] * 3,
    )(x, s1, b1, wqkv)


def _attn_call(q, k, v):
    def body(q_ref, k_ref, v_ref, o_ref, acc_ref, mx_ref, den_ref):
        i = pl.program_id(0)
        rloc = lax.broadcasted_iota(jnp.int32, (TT, TT), 0)
        cloc = lax.broadcasted_iota(jnp.int32, (TT, TT), 1)
        diag_ok = cloc <= rloc
        for h in range(N_HEADS):
            hs = slice(h * D_H, (h + 1) * D_H)
            qh = q_ref[:, hs]
            acc_ref[...] = jnp.zeros((TT, D_H), _F32)
            mx_ref[...] = jnp.full((TT, 1), -1e30, _F32)
            den_ref[...] = jnp.zeros((TT, 1), _F32)
            for j in range(NT):

                @pl.when(j <= i)
                def _(j=j):
                    kh = k_ref[j * TT:(j + 1) * TT, hs]
                    vh = v_ref[j * TT:(j + 1) * TT, hs]
                    s = lax.dot_general(qh, kh, (((1,), (1,)), ((), ())),
                                        preferred_element_type=_F32) * 0.125
                    s = jnp.where(jnp.logical_or(j < i, diag_ok),
                                  s, _F32(-1e9))
                    mx_old = mx_ref[...]
                    mx_new = jnp.maximum(
                        mx_old, jnp.max(s, axis=-1, keepdims=True))
                    e = jnp.exp(s - mx_new)
                    corr = jnp.exp(mx_old - mx_new)
                    den_ref[...] = (den_ref[...] * corr
                                    + jnp.sum(e, axis=-1, keepdims=True))
                    acc_ref[...] = (acc_ref[...] * corr
                                    + jnp.dot(e, vh,
                                              preferred_element_type=_F32))
                    mx_ref[...] = mx_new

            o_ref[:, hs] = acc_ref[...] / den_ref[...]

    return pl.pallas_call(
        body,
        grid=(NT,),
        in_specs=[
            pl.BlockSpec((TT, D_A), lambda i: (i, 0)),
            pl.BlockSpec((T, D_A), lambda i: (0, 0)),
            pl.BlockSpec((T, D_A), lambda i: (0, 0)),
        ],
        out_specs=pl.BlockSpec((TT, D_A), lambda i: (i, 0)),
        out_shape=jax.ShapeDtypeStruct((T, D_A), _F32),
        scratch_shapes=[
            pltpu.VMEM((TT, D_H), _F32),
            pltpu.VMEM((TT, 1), _F32),
            pltpu.VMEM((TT, 1), _F32),
        ],
    )(q, k, v)


def _woffn_call(x, attn, wo, s2, b2, w1, bb1, w2, bb2, want_zsum):
    def body(x_ref, a_ref, wo_ref, s_ref, b_ref, w1_ref, b1_ref,
             w2_ref, b2_ref, y_ref, *maybe_z):
        x1 = x_ref[...] + jnp.dot(a_ref[...], wo_ref[...],
                                  preferred_element_type=_F32)
        h2 = _ln_in(x1, s_ref[...], b_ref[...])
        ff = jax.nn.gelu(jnp.dot(h2, w1_ref[...],
                                 preferred_element_type=_F32) + b1_ref[...])
        y = x1 + jnp.dot(ff, w2_ref[...],
                         preferred_element_type=_F32) + b2_ref[...]
        y_ref[...] = y
        if maybe_z:
            maybe_z[0][...] = jnp.sum(y, axis=0, keepdims=True)[None]

    tile = lambda i: (i, 0)
    full = lambda i: (0, 0)
    out_specs = [pl.BlockSpec((TT, D_A), tile)]
    out_shape = [jax.ShapeDtypeStruct((T, D_A), _F32)]
    if want_zsum:
        out_specs.append(pl.BlockSpec((1, 1, D_A), lambda i: (i, 0, 0)))
        out_shape.append(jax.ShapeDtypeStruct((NT, 1, D_A), _F32))
    res = pl.pallas_call(
        body,
        grid=(NT,),
        in_specs=[
            pl.BlockSpec((TT, D_A), tile),
            pl.BlockSpec((TT, D_A), tile),
            pl.BlockSpec((D_A, D_A), full),
            pl.BlockSpec((1, D_A), full),
            pl.BlockSpec((1, D_A), full),
            pl.BlockSpec((D_A, D_FF), full),
            pl.BlockSpec((1, D_FF), full),
            pl.BlockSpec((D_FF, D_A), full),
            pl.BlockSpec((1, D_A), full),
        ],
        out_specs=out_specs,
        out_shape=out_shape,
    )(x, attn, wo, s2, b2, w1, bb1, w2, bb2)
    return res


def _route_m_call(zparts, wq2, wk2):
    def body(zp_ref, wq_ref, wk_ref, m_ref):
        z = jnp.sum(zp_ref[...][:, 0, :], axis=0, keepdims=True) * (1.0 / T)
        qf = jnp.dot(z, wq_ref[...], preferred_element_type=_F32)
        m = lax.dot_general(qf, wk_ref[...], (((1,), (1,)), ((), ())),
                            preferred_element_type=_F32)
        m_ref[...] = m * 0.125

    return pl.pallas_call(
        body,
        in_specs=[
            pl.BlockSpec((NT, 1, D_A), lambda: (0, 0, 0)),
            pl.BlockSpec((D_A, N_ASPECTS * D_K), lambda: (0, 0)),
            pl.BlockSpec((D_POOL, N_ASPECTS * D_K), lambda: (0, 0)),
        ],
        out_specs=pl.BlockSpec((1, D_POOL), lambda: (0, 0)),
        out_shape=jax.ShapeDtypeStruct((1, D_POOL), _F32),
    )(zparts, wq2, wk2)


def _combined_call(pool, m):
    def body(p_ref, m_ref, o_ref):
        o_ref[...] = lax.dot_general(
            p_ref[...], m_ref[...], (((1,), (1,)), ((), ())),
            preferred_element_type=_F32)

    return pl.pallas_call(
        body,
        grid=(NPT,),
        in_specs=[
            pl.BlockSpec((PT, D_POOL), lambda i: (i, 0)),
            pl.BlockSpec((1, D_POOL), lambda i: (0, 0)),
        ],
        out_specs=pl.BlockSpec((PT, 1), lambda i: (i, 0)),
        out_shape=jax.ShapeDtypeStruct((N_POOL, 1), _F32),
    )(pool, m)


def _topk_call(c2d, lam, warm):
    rows, cols = c2d.shape

    def body(c_ref, lam_ref, warm_ref, a_ref, i_ref):
        c = c_ref[...] * lam_ref[0]
        cmax = jnp.max(c)
        e = jnp.exp(c - cmax)
        soft = e / jnp.sum(e)
        flat = (lax.broadcasted_iota(jnp.int32, (rows, cols), 0) * cols
                + lax.broadcasted_iota(jnp.int32, (rows, cols), 1))
        cur = soft
        vals = []
        for kk in range(TOP_K):
            mx = jnp.max(cur)
            am = jnp.min(jnp.where(cur == mx, flat, jnp.int32(N_POOL)))
            vals.append(mx)
            i_ref[kk] = am
            cur = jnp.where(flat == am, _F32(-1.0), cur)
        vsum = vals[0]
        for kk in range(1, TOP_K):
            vsum = vsum + vals[kk]
        warmb = warm_ref[0] != 0
        for kk in range(TOP_K):
            a_ref[kk] = jnp.where(warmb, vals[kk],
                                  vals[kk] / (vsum + 1e-9))

    return pl.pallas_call(
        body,
        in_specs=[
            pl.BlockSpec((rows, cols), lambda: (0, 0)),
            pl.BlockSpec(memory_space=pltpu.SMEM),
            pl.BlockSpec(memory_space=pltpu.SMEM),
        ],
        out_specs=[
            pl.BlockSpec(memory_space=pltpu.SMEM),
            pl.BlockSpec(memory_space=pltpu.SMEM),
        ],
        out_shape=[
            jax.ShapeDtypeStruct((TOP_K,), _F32),
            jax.ShapeDtypeStruct((TOP_K,), jnp.int32),
        ],
    )(c2d, lam, warm)


def _wm_call(au, bv, alpha16, w_base, gamma):
    def body(au_ref, bv_ref, al_ref, wb_ref, g_ref, o_ref):
        delta = jnp.dot(au_ref[...] * al_ref[...], bv_ref[...],
                        preferred_element_type=_F32)
        o_ref[...] = wb_ref[...] + g_ref[0] * delta

    return pl.pallas_call(
        body,
        in_specs=[
            pl.BlockSpec((D_B, 2 * TOP_K), lambda: (0, 0)),
            pl.BlockSpec((2 * TOP_K, D_A), lambda: (0, 0)),
            pl.BlockSpec((1, 2 * TOP_K), lambda: (0, 0)),
            pl.BlockSpec((D_B, D_A), lambda: (0, 0)),
            pl.BlockSpec(memory_space=pltpu.SMEM),
        ],
        out_specs=pl.BlockSpec((D_B, D_A), lambda: (0, 0)),
        out_shape=jax.ShapeDtypeStruct((D_B, D_A), _F32),
    )(au, bv, alpha16, w_base, gamma)


def _hmid_call(h_a, wm, b_base, s, b):
    def body(x_ref, w_ref, bb_ref, s_ref, b_ref, o_ref):
        t = lax.dot_general(x_ref[...], w_ref[...],
                            (((1,), (1,)), ((), ())),
                            preferred_element_type=_F32) + bb_ref[...]
        o_ref[...] = _ln_in(t, s_ref[...], b_ref[...])

    tile = lambda i: (i, 0)
    full = lambda i: (0, 0)
    return pl.pallas_call(
        body,
        grid=(NT,),
        in_specs=[
            pl.BlockSpec((TT, D_A), tile),
            pl.BlockSpec((D_B, D_A), full),
            pl.BlockSpec((1, D_B), full),
            pl.BlockSpec((1, D_B), full),
            pl.BlockSpec((1, D_B), full),
        ],
        out_specs=pl.BlockSpec((TT, D_B), tile),
        out_shape=jax.ShapeDtypeStruct((T, D_B), _F32),
    )(h_a, wm, b_base, s, b)


def _lmhead_call(x, w):
    def body(x_ref, w_ref, o_ref):
        o_ref[...] = jnp.dot(x_ref[...], w_ref[...],
                             preferred_element_type=_F32)

    return pl.pallas_call(
        body,
        grid=(NVT,),
        in_specs=[
            pl.BlockSpec((T, D_B), lambda j: (0, 0)),
            pl.BlockSpec((D_B, VT), lambda j: (0, j)),
        ],
        out_specs=pl.BlockSpec((T, VT), lambda j: (0, j)),
        out_shape=jax.ShapeDtypeStruct((T, VOCAB), _F32),
    )(x, w)


def kernel(input_ids, lambda_val, is_warmup, embed_table, a_ln1_s, a_ln1_b,
           a_wqkv, a_wo, a_ln2_s, a_ln2_b, a_w1, a_b1, a_w2, a_b2,
           pool_vectors, w_key, w_query, w_base, b_base, gamma, asm_ln_s,
           asm_ln_b, b_ln1_s, b_ln1_b, b_wqkv, b_wo, b_ln2_s, b_ln2_b,
           b_w1, b_b1, b_w2, b_b2, lm_head_w):
    row2 = lambda a: jnp.asarray(a, _F32).reshape(1, -1)

    ids = input_ids.reshape(T).astype(jnp.int32)
    g = _embed_gather(embed_table, ids)
    pos = _pos_enc_const(T, D_A)

    q, k, v, x = _qkv_a_call(g, pos, row2(a_ln1_s), row2(a_ln1_b), a_wqkv)
    attn = _attn_call(q, k, v)
    h_a, zparts = _woffn_call(x, attn, a_wo, row2(a_ln2_s), row2(a_ln2_b),
                              a_w1, row2(a_b1), a_w2, row2(a_b2),
                              want_zsum=True)

    wq2 = w_query.transpose(1, 0, 2).reshape(D_A, N_ASPECTS * D_K)
    wk2 = w_key.transpose(1, 0, 2).reshape(D_POOL, N_ASPECTS * D_K)
    m = _route_m_call(zparts, wq2, wk2)
    combined = _combined_call(pool_vectors, m)
    lam = jnp.asarray(lambda_val, _F32).reshape(1)
    warm = jnp.asarray(is_warmup, jnp.int32).reshape(1)
    alphas, indices = _topk_call(combined.reshape(64, 128), lam, warm)

    gathered = _pool_gather(pool_vectors, indices)
    au = gathered[:, :D_B * R].reshape(TOP_K, D_B, R).transpose(1, 0, 2)
    au = au.reshape(D_B, TOP_K * R)
    bv = gathered[:, D_B * R:].reshape(TOP_K * R, D_A)
    alpha16 = jnp.repeat(alphas, R).reshape(1, TOP_K * R)
    wm = _wm_call(au, bv, alpha16, w_base, gamma.reshape(1))

    h_mid = _hmid_call(h_a, wm, row2(b_base), row2(asm_ln_s), row2(asm_ln_b))

    q2, k2, v2 = _qkv_b_call(h_mid, row2(b_ln1_s), row2(b_ln1_b), b_wqkv)
    attn2 = _attn_call(q2, k2, v2)
    [h_out] = _woffn_call(h_mid, attn2, b_wo, row2(b_ln2_s), row2(b_ln2_b),
                          b_w1, row2(b_b1), b_w2, row2(b_b2),
                          want_zsum=False)

    logits = _lmhead_call(h_out, lm_head_w)
    return logits.reshape(1, T, VOCAB)


# speedup vs baseline: 2.0682x; 2.0682x over previous
import functools

import jax
import jax.numpy as jnp
from jax import lax
from jax.experimental import pallas as pl
from jax.experimental.pallas import tpu as pltpu
from jax.experimental.pallas import tpu_sc as plsc

VOCAB = 32000
D_A = 768
D_B = 768
N_HEADS = 12
D_H = 64
D_FF = 3072
N_POOL = 8192
R = 2
TOP_K = 8
D_K = 64
N_ASPECTS = 4
T = 2048
D_POOL = R * (D_A + D_B)

TT = 256
NT = T // TT
PT = 1024
NPT = N_POOL // PT
VT = 640
NVT = VOCAB // VT

_F32 = jnp.float32


def _ln_in(x, s, b):
    m = jnp.mean(x, axis=-1, keepdims=True)
    v = jnp.mean((x - m) ** 2, axis=-1, keepdims=True)
    return (x - m) * lax.rsqrt(v + 1e-5) * s + b


def _pos_enc_const(seq_len, d_model):
    pos = jnp.arange(seq_len)[:, None]
    i = jnp.arange(d_model // 2)[None, :]
    angle = pos / 10000 ** (2 * i / d_model)
    enc = jnp.concatenate([jnp.sin(angle), jnp.cos(angle)], axis=-1)
    return enc[:, :d_model].astype(_F32)


def _embed_gather(table, idx):
    info = plsc.get_sparse_core_info()
    nc, ns = info.num_cores, info.num_subcores
    nw = nc * ns
    bpw = T // nw
    mesh = plsc.VectorSubcoreMesh(core_axis_name="c", subcore_axis_name="s")

    @functools.partial(
        pl.kernel, mesh=mesh,
        out_type=jax.ShapeDtypeStruct((T, D_A), _F32),
        scratch_types=[
            pltpu.VMEM((bpw,), jnp.int32),
            pltpu.VMEM((bpw, D_A), _F32),
            pltpu.SemaphoreType.DMA,
        ],
    )
    def k(table_hbm, idx_hbm, out_hbm, idx_v, rows_v, sem):
        wid = lax.axis_index("s") * nc + lax.axis_index("c")
        base = wid * bpw
        pltpu.sync_copy(idx_hbm.at[pl.ds(base, bpw)], idx_v)
        pltpu.async_copy(table_hbm.at[idx_v], rows_v, sem).wait()
        pltpu.sync_copy(rows_v, out_hbm.at[pl.ds(base, bpw)])

    return k(table, idx)


def _pool_gather(pool, idx):
    info = plsc.get_sparse_core_info()
    nc = info.num_cores
    mesh = plsc.VectorSubcoreMesh(core_axis_name="c", subcore_axis_name="s")

    @functools.partial(
        pl.kernel, mesh=mesh,
        out_type=jax.ShapeDtypeStruct((TOP_K, D_POOL), _F32),
        scratch_types=[
            pltpu.VMEM((TOP_K,), jnp.int32),
            pltpu.VMEM((TOP_K, D_POOL), _F32),
            pltpu.SemaphoreType.DMA,
        ],
    )
    def k(pool_hbm, idx_hbm, out_hbm, idx_v, rows_v, sem):
        wid = lax.axis_index("s") * nc + lax.axis_index("c")

        @pl.when(wid == 0)
        def _():
            pltpu.sync_copy(idx_hbm, idx_v)
            pltpu.async_copy(pool_hbm.at[idx_v], rows_v, sem).wait()
            pltpu.sync_copy(rows_v, out_hbm)

    return k(pool, idx)


def _qkv_a_call(g, pos, s1, b1, wqkv):
    def body(g_ref, p_ref, s_ref, b_ref, w_ref, q_ref, k_ref, v_ref, x_ref):
        x = g_ref[...] + p_ref[...]
        x_ref[...] = x
        h = _ln_in(x, s_ref[...], b_ref[...])
        qkv = jnp.dot(h, w_ref[...], preferred_element_type=_F32)
        q_ref[...] = qkv[:, :D_A]
        k_ref[...] = qkv[:, D_A:2 * D_A]
        v_ref[...] = qkv[:, 2 * D_A:]

    tile = lambda i: (i, 0)
    full = lambda i: (0, 0)
    return pl.pallas_call(
        body,
        grid=(NT,),
        in_specs=[
            pl.BlockSpec((TT, D_A), tile),
            pl.BlockSpec((TT, D_A), tile),
            pl.BlockSpec((1, D_A), full),
            pl.BlockSpec((1, D_A), full),
            pl.BlockSpec((D_A, 3 * D_A), full),
        ],
        out_specs=[
            pl.BlockSpec((TT, D_A), tile),
            pl.BlockSpec((TT, D_A), tile),
            pl.BlockSpec((TT, D_A), tile),
            pl.BlockSpec((TT, D_A), tile),
        ],
        out_shape=[jax.ShapeDtypeStruct((T, D_A), _F32)] * 4,
    )(g, pos, s1, b1, wqkv)


def _qkv_b_call(x, s1, b1, wqkv):
    def body(x_ref, s_ref, b_ref, w_ref, q_ref, k_ref, v_ref):
        h = _ln_in(x_ref[...], s_ref[...], b_ref[...])
        qkv = jnp.dot(h, w_ref[...], preferred_element_type=_F32)
        q_ref[...] = qkv[:, :D_A]
        k_ref[...] = qkv[:, D_A:2 * D_A]
        v_ref[...] = qkv[:, 2 * D_A:]

    tile = lambda i: (i, 0)
    full = lambda i: (0, 0)
    return pl.pallas_call(
        body,
        grid=(NT,),
        in_specs=[
            pl.BlockSpec((TT, D_A), tile),
            pl.BlockSpec((1, D_A), full),
            pl.BlockSpec((1, D_A), full),
            pl.BlockSpec((D_A, 3 * D_A), full),
        ],
        out_specs=[
            pl.BlockSpec((TT, D_A), tile),
            pl.BlockSpec((TT, D_A), tile),
            pl.BlockSpec((TT, D_A), tile),
        ],
        out_shape=[jax.ShapeDtypeStruct((T, D_A), _F32)] * 3,
    )(x, s1, b1, wqkv)


def _attn_call(q, k, v):
    def body(q_ref, k_ref, v_ref, o_ref):
        i = pl.program_id(0)
        row = i * TT + lax.broadcasted_iota(jnp.int32, (TT, T), 0)
        col = lax.broadcasted_iota(jnp.int32, (TT, T), 1)
        madd = jnp.where(col <= row, _F32(0.0), _F32(-1e9))
        outs = []
        for h in range(N_HEADS):
            hs = slice(h * D_H, (h + 1) * D_H)
            qh = q_ref[:, hs] * 0.125
            kh = k_ref[:, hs]
            vh = v_ref[:, hs]
            s = lax.dot_general(qh, kh, (((1,), (1,)), ((), ())),
                                preferred_element_type=_F32) + madd
            m = jnp.max(s, axis=-1, keepdims=True)
            e = jnp.exp(s - m)
            rden = 1.0 / jnp.sum(e, axis=-1, keepdims=True)
            outs.append(jnp.dot(e, vh, preferred_element_type=_F32) * rden)
        o_ref[...] = jnp.concatenate(outs, axis=1)

    return pl.pallas_call(
        body,
        grid=(NT,),
        in_specs=[
            pl.BlockSpec((TT, D_A), lambda i: (i, 0)),
            pl.BlockSpec((T, D_A), lambda i: (0, 0)),
            pl.BlockSpec((T, D_A), lambda i: (0, 0)),
        ],
        out_specs=pl.BlockSpec((TT, D_A), lambda i: (i, 0)),
        out_shape=jax.ShapeDtypeStruct((T, D_A), _F32),
    )(q, k, v)


def _woffn_call(x, attn, wo, s2, b2, w1, bb1, w2, bb2, want_zsum):
    def body(x_ref, a_ref, wo_ref, s_ref, b_ref, w1_ref, b1_ref,
             w2_ref, b2_ref, y_ref, *maybe_z):
        x1 = x_ref[...] + jnp.dot(a_ref[...], wo_ref[...],
                                  preferred_element_type=_F32)
        h2 = _ln_in(x1, s_ref[...], b_ref[...])
        ff = jax.nn.gelu(jnp.dot(h2, w1_ref[...],
                                 preferred_element_type=_F32) + b1_ref[...])
        y = x1 + jnp.dot(ff, w2_ref[...],
                         preferred_element_type=_F32) + b2_ref[...]
        y_ref[...] = y
        if maybe_z:
            maybe_z[0][...] = jnp.sum(y, axis=0, keepdims=True)[None]

    tile = lambda i: (i, 0)
    full = lambda i: (0, 0)
    out_specs = [pl.BlockSpec((TT, D_A), tile)]
    out_shape = [jax.ShapeDtypeStruct((T, D_A), _F32)]
    if want_zsum:
        out_specs.append(pl.BlockSpec((1, 1, D_A), lambda i: (i, 0, 0)))
        out_shape.append(jax.ShapeDtypeStruct((NT, 1, D_A), _F32))
    res = pl.pallas_call(
        body,
        grid=(NT,),
        in_specs=[
            pl.BlockSpec((TT, D_A), tile),
            pl.BlockSpec((TT, D_A), tile),
            pl.BlockSpec((D_A, D_A), full),
            pl.BlockSpec((1, D_A), full),
            pl.BlockSpec((1, D_A), full),
            pl.BlockSpec((D_A, D_FF), full),
            pl.BlockSpec((1, D_FF), full),
            pl.BlockSpec((D_FF, D_A), full),
            pl.BlockSpec((1, D_A), full),
        ],
        out_specs=out_specs,
        out_shape=out_shape,
    )(x, attn, wo, s2, b2, w1, bb1, w2, bb2)
    return res


def _route_m_call(zparts, wq2, wk2):
    def body(zp_ref, wq_ref, wk_ref, m_ref):
        z = jnp.sum(zp_ref[...][:, 0, :], axis=0, keepdims=True) * (1.0 / T)
        qf = jnp.dot(z, wq_ref[...], preferred_element_type=_F32)
        m = lax.dot_general(qf, wk_ref[...], (((1,), (1,)), ((), ())),
                            preferred_element_type=_F32)
        m_ref[...] = m * 0.125

    return pl.pallas_call(
        body,
        in_specs=[
            pl.BlockSpec((NT, 1, D_A), lambda: (0, 0, 0)),
            pl.BlockSpec((D_A, N_ASPECTS * D_K), lambda: (0, 0)),
            pl.BlockSpec((D_POOL, N_ASPECTS * D_K), lambda: (0, 0)),
        ],
        out_specs=pl.BlockSpec((1, D_POOL), lambda: (0, 0)),
        out_shape=jax.ShapeDtypeStruct((1, D_POOL), _F32),
    )(zparts, wq2, wk2)


def _combined_call(pool, m):
    def body(p_ref, m_ref, o_ref):
        o_ref[...] = lax.dot_general(
            p_ref[...], m_ref[...], (((1,), (1,)), ((), ())),
            preferred_element_type=_F32)

    return pl.pallas_call(
        body,
        grid=(NPT,),
        in_specs=[
            pl.BlockSpec((PT, D_POOL), lambda i: (i, 0)),
            pl.BlockSpec((1, D_POOL), lambda i: (0, 0)),
        ],
        out_specs=pl.BlockSpec((PT, 1), lambda i: (i, 0)),
        out_shape=jax.ShapeDtypeStruct((N_POOL, 1), _F32),
    )(pool, m)


def _topk_call(c2d, lam, warm):
    rows, cols = c2d.shape

    def body(c_ref, lam_ref, warm_ref, a_ref, i_ref):
        c = c_ref[...] * lam_ref[0]
        cmax = jnp.max(c)
        e = jnp.exp(c - cmax)
        soft = e / jnp.sum(e)
        flat = (lax.broadcasted_iota(jnp.int32, (rows, cols), 0) * cols
                + lax.broadcasted_iota(jnp.int32, (rows, cols), 1))
        cur = soft
        vals = []
        for kk in range(TOP_K):
            mx = jnp.max(cur)
            am = jnp.min(jnp.where(cur == mx, flat, jnp.int32(N_POOL)))
            vals.append(mx)
            i_ref[kk] = am
            cur = jnp.where(flat == am, _F32(-1.0), cur)
        vsum = vals[0]
        for kk in range(1, TOP_K):
            vsum = vsum + vals[kk]
        warmb = warm_ref[0] != 0
        for kk in range(TOP_K):
            a_ref[kk] = jnp.where(warmb, vals[kk],
                                  vals[kk] / (vsum + 1e-9))

    return pl.pallas_call(
        body,
        in_specs=[
            pl.BlockSpec((rows, cols), lambda: (0, 0)),
            pl.BlockSpec(memory_space=pltpu.SMEM),
            pl.BlockSpec(memory_space=pltpu.SMEM),
        ],
        out_specs=[
            pl.BlockSpec(memory_space=pltpu.SMEM),
            pl.BlockSpec(memory_space=pltpu.SMEM),
        ],
        out_shape=[
            jax.ShapeDtypeStruct((TOP_K,), _F32),
            jax.ShapeDtypeStruct((TOP_K,), jnp.int32),
        ],
    )(c2d, lam, warm)


def _wm_call(au, bv, alpha16, w_base, gamma):
    def body(au_ref, bv_ref, al_ref, wb_ref, g_ref, o_ref):
        delta = jnp.dot(au_ref[...] * al_ref[...], bv_ref[...],
                        preferred_element_type=_F32)
        o_ref[...] = wb_ref[...] + g_ref[0] * delta

    return pl.pallas_call(
        body,
        in_specs=[
            pl.BlockSpec((D_B, 2 * TOP_K), lambda: (0, 0)),
            pl.BlockSpec((2 * TOP_K, D_A), lambda: (0, 0)),
            pl.BlockSpec((1, 2 * TOP_K), lambda: (0, 0)),
            pl.BlockSpec((D_B, D_A), lambda: (0, 0)),
            pl.BlockSpec(memory_space=pltpu.SMEM),
        ],
        out_specs=pl.BlockSpec((D_B, D_A), lambda: (0, 0)),
        out_shape=jax.ShapeDtypeStruct((D_B, D_A), _F32),
    )(au, bv, alpha16, w_base, gamma)


def _hmid_call(h_a, wm, b_base, s, b):
    def body(x_ref, w_ref, bb_ref, s_ref, b_ref, o_ref):
        t = lax.dot_general(x_ref[...], w_ref[...],
                            (((1,), (1,)), ((), ())),
                            preferred_element_type=_F32) + bb_ref[...]
        o_ref[...] = _ln_in(t, s_ref[...], b_ref[...])

    tile = lambda i: (i, 0)
    full = lambda i: (0, 0)
    return pl.pallas_call(
        body,
        grid=(NT,),
        in_specs=[
            pl.BlockSpec((TT, D_A), tile),
            pl.BlockSpec((D_B, D_A), full),
            pl.BlockSpec((1, D_B), full),
            pl.BlockSpec((1, D_B), full),
            pl.BlockSpec((1, D_B), full),
        ],
        out_specs=pl.BlockSpec((TT, D_B), tile),
        out_shape=jax.ShapeDtypeStruct((T, D_B), _F32),
    )(h_a, wm, b_base, s, b)


def _lmhead_call(x, w):
    def body(x_ref, w_ref, o_ref):
        o_ref[...] = jnp.dot(x_ref[...], w_ref[...],
                             preferred_element_type=_F32)

    return pl.pallas_call(
        body,
        grid=(NVT,),
        in_specs=[
            pl.BlockSpec((T, D_B), lambda j: (0, 0)),
            pl.BlockSpec((D_B, VT), lambda j: (0, j)),
        ],
        out_specs=pl.BlockSpec((T, VT), lambda j: (0, j)),
        out_shape=jax.ShapeDtypeStruct((T, VOCAB), _F32),
    )(x, w)


def kernel(input_ids, lambda_val, is_warmup, embed_table, a_ln1_s, a_ln1_b,
           a_wqkv, a_wo, a_ln2_s, a_ln2_b, a_w1, a_b1, a_w2, a_b2,
           pool_vectors, w_key, w_query, w_base, b_base, gamma, asm_ln_s,
           asm_ln_b, b_ln1_s, b_ln1_b, b_wqkv, b_wo, b_ln2_s, b_ln2_b,
           b_w1, b_b1, b_w2, b_b2, lm_head_w):
    row2 = lambda a: jnp.asarray(a, _F32).reshape(1, -1)

    ids = input_ids.reshape(T).astype(jnp.int32)
    g = _embed_gather(embed_table, ids)
    pos = _pos_enc_const(T, D_A)

    q, k, v, x = _qkv_a_call(g, pos, row2(a_ln1_s), row2(a_ln1_b), a_wqkv)
    attn = _attn_call(q, k, v)
    h_a, zparts = _woffn_call(x, attn, a_wo, row2(a_ln2_s), row2(a_ln2_b),
                              a_w1, row2(a_b1), a_w2, row2(a_b2),
                              want_zsum=True)

    wq2 = w_query.transpose(1, 0, 2).reshape(D_A, N_ASPECTS * D_K)
    wk2 = w_key.transpose(1, 0, 2).reshape(D_POOL, N_ASPECTS * D_K)
    m = _route_m_call(zparts, wq2, wk2)
    combined = _combined_call(pool_vectors, m)
    lam = jnp.asarray(lambda_val, _F32).reshape(1)
    warm = jnp.asarray(is_warmup, jnp.int32).reshape(1)
    alphas, indices = _topk_call(combined.reshape(64, 128), lam, warm)

    gathered = _pool_gather(pool_vectors, indices)
    au = gathered[:, :D_B * R].reshape(TOP_K, D_B, R).transpose(1, 0, 2)
    au = au.reshape(D_B, TOP_K * R)
    bv = gathered[:, D_B * R:].reshape(TOP_K * R, D_A)
    alpha16 = jnp.repeat(alphas, R).reshape(1, TOP_K * R)
    wm = _wm_call(au, bv, alpha16, w_base, gamma.reshape(1))

    h_mid = _hmid_call(h_a, wm, row2(b_base), row2(asm_ln_s), row2(asm_ln_b))

    q2, k2, v2 = _qkv_b_call(h_mid, row2(b_ln1_s), row2(b_ln1_b), b_wqkv)
    attn2 = _attn_call(q2, k2, v2)
    [h_out] = _woffn_call(h_mid, attn2, b_wo, row2(b_ln2_s), row2(b_ln2_b),
                          b_w1, row2(b_b1), b_w2, row2(b_b2),
                          want_zsum=False)

    logits = _lmhead_call(h_out, lm_head_w)
    return logits.reshape(1, T, VOCAB)


# speedup vs baseline: 2.1645x; 1.0466x over previous
import functools

import jax
import jax.numpy as jnp
from jax import lax
from jax.experimental import pallas as pl
from jax.experimental.pallas import tpu as pltpu
from jax.experimental.pallas import tpu_sc as plsc

VOCAB = 32000
D_A = 768
D_B = 768
N_HEADS = 12
D_H = 64
D_FF = 3072
N_POOL = 8192
R = 2
TOP_K = 8
D_K = 64
N_ASPECTS = 4
T = 2048
D_POOL = R * (D_A + D_B)

TT = 256
NT = T // TT
PT = 1024
NPT = N_POOL // PT
VT = 1280
NVT = VOCAB // VT

_F32 = jnp.float32


def _ln_in(x, s, b):
    m = jnp.mean(x, axis=-1, keepdims=True)
    v = jnp.mean((x - m) ** 2, axis=-1, keepdims=True)
    return (x - m) * lax.rsqrt(v + 1e-5) * s + b


def _pos_enc_const(seq_len, d_model):
    pos = jnp.arange(seq_len)[:, None]
    i = jnp.arange(d_model // 2)[None, :]
    angle = pos / 10000 ** (2 * i / d_model)
    enc = jnp.concatenate([jnp.sin(angle), jnp.cos(angle)], axis=-1)
    return enc[:, :d_model].astype(_F32)


def _embed_gather(table, idx):
    info = plsc.get_sparse_core_info()
    nc, ns = info.num_cores, info.num_subcores
    nw = nc * ns
    bpw = T // nw
    mesh = plsc.VectorSubcoreMesh(core_axis_name="c", subcore_axis_name="s")

    @functools.partial(
        pl.kernel, mesh=mesh,
        out_type=jax.ShapeDtypeStruct((T, D_A), _F32),
        scratch_types=[
            pltpu.VMEM((bpw,), jnp.int32),
            pltpu.VMEM((bpw, D_A), _F32),
            pltpu.SemaphoreType.DMA,
        ],
    )
    def k(table_hbm, idx_hbm, out_hbm, idx_v, rows_v, sem):
        wid = lax.axis_index("s") * nc + lax.axis_index("c")
        base = wid * bpw
        pltpu.sync_copy(idx_hbm.at[pl.ds(base, bpw)], idx_v)
        pltpu.async_copy(table_hbm.at[idx_v], rows_v, sem).wait()
        pltpu.sync_copy(rows_v, out_hbm.at[pl.ds(base, bpw)])

    return k(table, idx)


def _pool_gather(pool, idx):
    info = plsc.get_sparse_core_info()
    nc = info.num_cores
    mesh = plsc.VectorSubcoreMesh(core_axis_name="c", subcore_axis_name="s")

    @functools.partial(
        pl.kernel, mesh=mesh,
        out_type=jax.ShapeDtypeStruct((TOP_K, D_POOL), _F32),
        scratch_types=[
            pltpu.VMEM((TOP_K,), jnp.int32),
            pltpu.VMEM((TOP_K, D_POOL), _F32),
            pltpu.SemaphoreType.DMA,
        ],
    )
    def k(pool_hbm, idx_hbm, out_hbm, idx_v, rows_v, sem):
        wid = lax.axis_index("s") * nc + lax.axis_index("c")

        @pl.when(wid == 0)
        def _():
            pltpu.sync_copy(idx_hbm, idx_v)
            pltpu.async_copy(pool_hbm.at[idx_v], rows_v, sem).wait()
            pltpu.sync_copy(rows_v, out_hbm)

    return k(pool, idx)


def _qkv_a_call(g, pos, s1, b1, wqkv):
    def body(g_ref, p_ref, s_ref, b_ref, w_ref, q_ref, k_ref, v_ref, x_ref):
        x = g_ref[...] + p_ref[...]
        x_ref[...] = x
        h = _ln_in(x, s_ref[...], b_ref[...])
        qkv = jnp.dot(h, w_ref[...], preferred_element_type=_F32)
        q_ref[...] = qkv[:, :D_A]
        k_ref[...] = qkv[:, D_A:2 * D_A]
        v_ref[...] = qkv[:, 2 * D_A:]

    tile = lambda i: (i, 0)
    full = lambda i: (0, 0)
    return pl.pallas_call(
        body,
        grid=(NT,),
        in_specs=[
            pl.BlockSpec((TT, D_A), tile),
            pl.BlockSpec((TT, D_A), tile),
            pl.BlockSpec((1, D_A), full),
            pl.BlockSpec((1, D_A), full),
            pl.BlockSpec((D_A, 3 * D_A), full),
        ],
        out_specs=[
            pl.BlockSpec((TT, D_A), tile),
            pl.BlockSpec((TT, D_A), tile),
            pl.BlockSpec((TT, D_A), tile),
            pl.BlockSpec((TT, D_A), tile),
        ],
        out_shape=[jax.ShapeDtypeStruct((T, D_A), _F32)] * 4,
    )(g, pos, s1, b1, wqkv)


def _qkv_b_call(x, s1, b1, wqkv):
    def body(x_ref, s_ref, b_ref, w_ref, q_ref, k_ref, v_ref):
        h = _ln_in(x_ref[...], s_ref[...], b_ref[...])
        qkv = jnp.dot(h, w_ref[...], preferred_element_type=_F32)
        q_ref[...] = qkv[:, :D_A]
        k_ref[...] = qkv[:, D_A:2 * D_A]
        v_ref[...] = qkv[:, 2 * D_A:]

    tile = lambda i: (i, 0)
    full = lambda i: (0, 0)
    return pl.pallas_call(
        body,
        grid=(NT,),
        in_specs=[
            pl.BlockSpec((TT, D_A), tile),
            pl.BlockSpec((1, D_A), full),
            pl.BlockSpec((1, D_A), full),
            pl.BlockSpec((D_A, 3 * D_A), full),
        ],
        out_specs=[
            pl.BlockSpec((TT, D_A), tile),
            pl.BlockSpec((TT, D_A), tile),
            pl.BlockSpec((TT, D_A), tile),
        ],
        out_shape=[jax.ShapeDtypeStruct((T, D_A), _F32)] * 3,
    )(x, s1, b1, wqkv)


def _attn_call(q, k, v):
    def body(q_ref, k_ref, v_ref, o_ref):
        i = pl.program_id(0)
        row = i * TT + lax.broadcasted_iota(jnp.int32, (TT, T), 0)
        col = lax.broadcasted_iota(jnp.int32, (TT, T), 1)
        madd = jnp.where(col <= row, _F32(0.0), _F32(-1e9))
        outs = []
        for h in range(N_HEADS):
            hs = slice(h * D_H, (h + 1) * D_H)
            qh = q_ref[:, hs] * 0.125
            kh = k_ref[:, hs]
            vh = v_ref[:, hs]
            s = lax.dot_general(qh, kh, (((1,), (1,)), ((), ())),
                                preferred_element_type=_F32) + madd
            m = jnp.max(s, axis=-1, keepdims=True)
            e = jnp.exp(s - m)
            rden = 1.0 / jnp.sum(e, axis=-1, keepdims=True)
            outs.append(jnp.dot(e, vh, preferred_element_type=_F32) * rden)
        o_ref[...] = jnp.concatenate(outs, axis=1)

    return pl.pallas_call(
        body,
        grid=(NT,),
        in_specs=[
            pl.BlockSpec((TT, D_A), lambda i: (i, 0)),
            pl.BlockSpec((T, D_A), lambda i: (0, 0)),
            pl.BlockSpec((T, D_A), lambda i: (0, 0)),
        ],
        out_specs=pl.BlockSpec((TT, D_A), lambda i: (i, 0)),
        out_shape=jax.ShapeDtypeStruct((T, D_A), _F32),
    )(q, k, v)


def _woffn_call(x, attn, wo, s2, b2, w1, bb1, w2, bb2, want_zsum):
    def body(x_ref, a_ref, wo_ref, s_ref, b_ref, w1_ref, b1_ref,
             w2_ref, b2_ref, y_ref, *maybe_z):
        x1 = x_ref[...] + jnp.dot(a_ref[...], wo_ref[...],
                                  preferred_element_type=_F32)
        h2 = _ln_in(x1, s_ref[...], b_ref[...])
        ff = jax.nn.gelu(jnp.dot(h2, w1_ref[...],
                                 preferred_element_type=_F32) + b1_ref[...])
        y = x1 + jnp.dot(ff, w2_ref[...],
                         preferred_element_type=_F32) + b2_ref[...]
        y_ref[...] = y
        if maybe_z:
            maybe_z[0][...] = jnp.sum(y, axis=0, keepdims=True)[None]

    tile = lambda i: (i, 0)
    full = lambda i: (0, 0)
    out_specs = [pl.BlockSpec((TT, D_A), tile)]
    out_shape = [jax.ShapeDtypeStruct((T, D_A), _F32)]
    if want_zsum:
        out_specs.append(pl.BlockSpec((1, 1, D_A), lambda i: (i, 0, 0)))
        out_shape.append(jax.ShapeDtypeStruct((NT, 1, D_A), _F32))
    res = pl.pallas_call(
        body,
        grid=(NT,),
        in_specs=[
            pl.BlockSpec((TT, D_A), tile),
            pl.BlockSpec((TT, D_A), tile),
            pl.BlockSpec((D_A, D_A), full),
            pl.BlockSpec((1, D_A), full),
            pl.BlockSpec((1, D_A), full),
            pl.BlockSpec((D_A, D_FF), full),
            pl.BlockSpec((1, D_FF), full),
            pl.BlockSpec((D_FF, D_A), full),
            pl.BlockSpec((1, D_A), full),
        ],
        out_specs=out_specs,
        out_shape=out_shape,
    )(x, attn, wo, s2, b2, w1, bb1, w2, bb2)
    return res


def _route_m_call(zparts, wq2, wk2):
    def body(zp_ref, wq_ref, wk_ref, m_ref):
        z = jnp.sum(zp_ref[...][:, 0, :], axis=0, keepdims=True) * (1.0 / T)
        qf = jnp.dot(z, wq_ref[...], preferred_element_type=_F32)
        m = lax.dot_general(qf, wk_ref[...], (((1,), (1,)), ((), ())),
                            preferred_element_type=_F32)
        m_ref[...] = m * 0.125

    return pl.pallas_call(
        body,
        in_specs=[
            pl.BlockSpec((NT, 1, D_A), lambda: (0, 0, 0)),
            pl.BlockSpec((D_A, N_ASPECTS * D_K), lambda: (0, 0)),
            pl.BlockSpec((D_POOL, N_ASPECTS * D_K), lambda: (0, 0)),
        ],
        out_specs=pl.BlockSpec((1, D_POOL), lambda: (0, 0)),
        out_shape=jax.ShapeDtypeStruct((1, D_POOL), _F32),
    )(zparts, wq2, wk2)


def _combined_call(pool, m):
    def body(p_ref, m_ref, o_ref):
        o_ref[...] = lax.dot_general(
            p_ref[...], m_ref[...], (((1,), (1,)), ((), ())),
            preferred_element_type=_F32)

    return pl.pallas_call(
        body,
        grid=(NPT,),
        in_specs=[
            pl.BlockSpec((PT, D_POOL), lambda i: (i, 0)),
            pl.BlockSpec((1, D_POOL), lambda i: (0, 0)),
        ],
        out_specs=pl.BlockSpec((PT, 1), lambda i: (i, 0)),
        out_shape=jax.ShapeDtypeStruct((N_POOL, 1), _F32),
    )(pool, m)


def _topk_call(c2d, lam, warm):
    rows, cols = c2d.shape

    def body(c_ref, lam_ref, warm_ref, a_ref, i_ref):
        c = c_ref[...] * lam_ref[0]
        cmax = jnp.max(c)
        e = jnp.exp(c - cmax)
        soft = e / jnp.sum(e)
        flat = (lax.broadcasted_iota(jnp.int32, (rows, cols), 0) * cols
                + lax.broadcasted_iota(jnp.int32, (rows, cols), 1))
        cur = soft
        vals = []
        for kk in range(TOP_K):
            mx = jnp.max(cur)
            am = jnp.min(jnp.where(cur == mx, flat, jnp.int32(N_POOL)))
            vals.append(mx)
            i_ref[kk] = am
            cur = jnp.where(flat == am, _F32(-1.0), cur)
        vsum = vals[0]
        for kk in range(1, TOP_K):
            vsum = vsum + vals[kk]
        warmb = warm_ref[0] != 0
        for kk in range(TOP_K):
            a_ref[kk] = jnp.where(warmb, vals[kk],
                                  vals[kk] / (vsum + 1e-9))

    return pl.pallas_call(
        body,
        in_specs=[
            pl.BlockSpec((rows, cols), lambda: (0, 0)),
            pl.BlockSpec(memory_space=pltpu.SMEM),
            pl.BlockSpec(memory_space=pltpu.SMEM),
        ],
        out_specs=[
            pl.BlockSpec(memory_space=pltpu.SMEM),
            pl.BlockSpec(memory_space=pltpu.SMEM),
        ],
        out_shape=[
            jax.ShapeDtypeStruct((TOP_K,), _F32),
            jax.ShapeDtypeStruct((TOP_K,), jnp.int32),
        ],
    )(c2d, lam, warm)


def _wm_call(au, bv, alpha16, w_base, gamma):
    def body(au_ref, bv_ref, al_ref, wb_ref, g_ref, o_ref):
        delta = jnp.dot(au_ref[...] * al_ref[...], bv_ref[...],
                        preferred_element_type=_F32)
        o_ref[...] = wb_ref[...] + g_ref[0] * delta

    return pl.pallas_call(
        body,
        in_specs=[
            pl.BlockSpec((D_B, 2 * TOP_K), lambda: (0, 0)),
            pl.BlockSpec((2 * TOP_K, D_A), lambda: (0, 0)),
            pl.BlockSpec((1, 2 * TOP_K), lambda: (0, 0)),
            pl.BlockSpec((D_B, D_A), lambda: (0, 0)),
            pl.BlockSpec(memory_space=pltpu.SMEM),
        ],
        out_specs=pl.BlockSpec((D_B, D_A), lambda: (0, 0)),
        out_shape=jax.ShapeDtypeStruct((D_B, D_A), _F32),
    )(au, bv, alpha16, w_base, gamma)


def _hmid_call(h_a, wm, b_base, s, b):
    def body(x_ref, w_ref, bb_ref, s_ref, b_ref, o_ref):
        t = lax.dot_general(x_ref[...], w_ref[...],
                            (((1,), (1,)), ((), ())),
                            preferred_element_type=_F32) + bb_ref[...]
        o_ref[...] = _ln_in(t, s_ref[...], b_ref[...])

    tile = lambda i: (i, 0)
    full = lambda i: (0, 0)
    return pl.pallas_call(
        body,
        grid=(NT,),
        in_specs=[
            pl.BlockSpec((TT, D_A), tile),
            pl.BlockSpec((D_B, D_A), full),
            pl.BlockSpec((1, D_B), full),
            pl.BlockSpec((1, D_B), full),
            pl.BlockSpec((1, D_B), full),
        ],
        out_specs=pl.BlockSpec((TT, D_B), tile),
        out_shape=jax.ShapeDtypeStruct((T, D_B), _F32),
    )(h_a, wm, b_base, s, b)


def _lmhead_call(x, w):
    def body(x_ref, w_ref, o_ref):
        o_ref[...] = jnp.dot(x_ref[...], w_ref[...],
                             preferred_element_type=_F32)

    return pl.pallas_call(
        body,
        grid=(NVT,),
        in_specs=[
            pl.BlockSpec((T, D_B), lambda j: (0, 0)),
            pl.BlockSpec((D_B, VT), lambda j: (0, j)),
        ],
        out_specs=pl.BlockSpec((T, VT), lambda j: (0, j)),
        out_shape=jax.ShapeDtypeStruct((T, VOCAB), _F32),
    )(x, w)


def kernel(input_ids, lambda_val, is_warmup, embed_table, a_ln1_s, a_ln1_b,
           a_wqkv, a_wo, a_ln2_s, a_ln2_b, a_w1, a_b1, a_w2, a_b2,
           pool_vectors, w_key, w_query, w_base, b_base, gamma, asm_ln_s,
           asm_ln_b, b_ln1_s, b_ln1_b, b_wqkv, b_wo, b_ln2_s, b_ln2_b,
           b_w1, b_b1, b_w2, b_b2, lm_head_w):
    row2 = lambda a: jnp.asarray(a, _F32).reshape(1, -1)

    ids = input_ids.reshape(T).astype(jnp.int32)
    g = _embed_gather(embed_table, ids)
    pos = _pos_enc_const(T, D_A)

    q, k, v, x = _qkv_a_call(g, pos, row2(a_ln1_s), row2(a_ln1_b), a_wqkv)
    attn = _attn_call(q, k, v)
    h_a, zparts = _woffn_call(x, attn, a_wo, row2(a_ln2_s), row2(a_ln2_b),
                              a_w1, row2(a_b1), a_w2, row2(a_b2),
                              want_zsum=True)

    wq2 = w_query.transpose(1, 0, 2).reshape(D_A, N_ASPECTS * D_K)
    wk2 = w_key.transpose(1, 0, 2).reshape(D_POOL, N_ASPECTS * D_K)
    m = _route_m_call(zparts, wq2, wk2)
    combined = _combined_call(pool_vectors, m)
    lam = jnp.asarray(lambda_val, _F32).reshape(1)
    warm = jnp.asarray(is_warmup, jnp.int32).reshape(1)
    alphas, indices = _topk_call(combined.reshape(64, 128), lam, warm)

    gathered = _pool_gather(pool_vectors, indices)
    au = gathered[:, :D_B * R].reshape(TOP_K, D_B, R).transpose(1, 0, 2)
    au = au.reshape(D_B, TOP_K * R)
    bv = gathered[:, D_B * R:].reshape(TOP_K * R, D_A)
    alpha16 = jnp.repeat(alphas, R).reshape(1, TOP_K * R)
    wm = _wm_call(au, bv, alpha16, w_base, gamma.reshape(1))

    h_mid = _hmid_call(h_a, wm, row2(b_base), row2(asm_ln_s), row2(asm_ln_b))

    q2, k2, v2 = _qkv_b_call(h_mid, row2(b_ln1_s), row2(b_ln1_b), b_wqkv)
    attn2 = _attn_call(q2, k2, v2)
    [h_out] = _woffn_call(h_mid, attn2, b_wo, row2(b_ln2_s), row2(b_ln2_b),
                          b_w1, row2(b_b1), b_w2, row2(b_b2),
                          want_zsum=False)

    logits = _lmhead_call(h_out, lm_head_w)
    return logits.reshape(1, T, VOCAB)


# speedup vs baseline: 2.1671x; 1.0012x over previous
import functools

import jax
import jax.numpy as jnp
from jax import lax
from jax.experimental import pallas as pl
from jax.experimental.pallas import tpu as pltpu
from jax.experimental.pallas import tpu_sc as plsc

VOCAB = 32000
D_A = 768
D_B = 768
N_HEADS = 12
D_H = 64
D_FF = 3072
N_POOL = 8192
R = 2
TOP_K = 8
D_K = 64
N_ASPECTS = 4
T = 2048
D_POOL = R * (D_A + D_B)

TT = 256
NT = T // TT
PT = 1024
NPT = N_POOL // PT
VT = 1280
NVT = VOCAB // VT

_F32 = jnp.float32


def _ln_in(x, s, b):
    m = jnp.mean(x, axis=-1, keepdims=True)
    v = jnp.mean((x - m) ** 2, axis=-1, keepdims=True)
    return (x - m) * lax.rsqrt(v + 1e-5) * s + b


def _pos_enc_const(seq_len, d_model):
    pos = jnp.arange(seq_len)[:, None]
    i = jnp.arange(d_model // 2)[None, :]
    angle = pos / 10000 ** (2 * i / d_model)
    enc = jnp.concatenate([jnp.sin(angle), jnp.cos(angle)], axis=-1)
    return enc[:, :d_model].astype(_F32)


def _embed_gather(table, idx):
    info = plsc.get_sparse_core_info()
    nc, ns = info.num_cores, info.num_subcores
    nw = nc * ns
    bpw = T // nw
    mesh = plsc.VectorSubcoreMesh(core_axis_name="c", subcore_axis_name="s")

    @functools.partial(
        pl.kernel, mesh=mesh,
        out_type=jax.ShapeDtypeStruct((T, D_A), _F32),
        scratch_types=[
            pltpu.VMEM((bpw,), jnp.int32),
            pltpu.VMEM((bpw, D_A), _F32),
            pltpu.SemaphoreType.DMA,
        ],
    )
    def k(table_hbm, idx_hbm, out_hbm, idx_v, rows_v, sem):
        wid = lax.axis_index("s") * nc + lax.axis_index("c")
        base = wid * bpw
        pltpu.sync_copy(idx_hbm.at[pl.ds(base, bpw)], idx_v)
        pltpu.async_copy(table_hbm.at[idx_v], rows_v, sem).wait()
        pltpu.sync_copy(rows_v, out_hbm.at[pl.ds(base, bpw)])

    return k(table, idx)


def _pool_gather(pool, idx):
    info = plsc.get_sparse_core_info()
    nc = info.num_cores
    mesh = plsc.VectorSubcoreMesh(core_axis_name="c", subcore_axis_name="s")

    @functools.partial(
        pl.kernel, mesh=mesh,
        out_type=jax.ShapeDtypeStruct((TOP_K, D_POOL), _F32),
        scratch_types=[
            pltpu.VMEM((TOP_K,), jnp.int32),
            pltpu.VMEM((TOP_K, D_POOL), _F32),
            pltpu.SemaphoreType.DMA,
        ],
    )
    def k(pool_hbm, idx_hbm, out_hbm, idx_v, rows_v, sem):
        wid = lax.axis_index("s") * nc + lax.axis_index("c")

        @pl.when(wid == 0)
        def _():
            pltpu.sync_copy(idx_hbm, idx_v)
            pltpu.async_copy(pool_hbm.at[idx_v], rows_v, sem).wait()
            pltpu.sync_copy(rows_v, out_hbm)

    return k(pool, idx)


def _qkv_a_call(g, pos, s1, b1, wqkv):
    def body(g_ref, p_ref, s_ref, b_ref, w_ref, q_ref, k_ref, v_ref, x_ref):
        x = g_ref[...] + p_ref[...]
        x_ref[...] = x
        h = _ln_in(x, s_ref[...], b_ref[...])
        qkv = jnp.dot(h, w_ref[...], preferred_element_type=_F32)
        q_ref[...] = qkv[:, :D_A]
        k_ref[...] = qkv[:, D_A:2 * D_A]
        v_ref[...] = qkv[:, 2 * D_A:]

    tile = lambda i: (i, 0)
    full = lambda i: (0, 0)
    return pl.pallas_call(
        body,
        grid=(NT,),
        in_specs=[
            pl.BlockSpec((TT, D_A), tile),
            pl.BlockSpec((TT, D_A), tile),
            pl.BlockSpec((1, D_A), full),
            pl.BlockSpec((1, D_A), full),
            pl.BlockSpec((D_A, 3 * D_A), full),
        ],
        out_specs=[
            pl.BlockSpec((TT, D_A), tile),
            pl.BlockSpec((TT, D_A), tile),
            pl.BlockSpec((TT, D_A), tile),
            pl.BlockSpec((TT, D_A), tile),
        ],
        out_shape=[jax.ShapeDtypeStruct((T, D_A), _F32)] * 4,
    )(g, pos, s1, b1, wqkv)


def _qkv_b_call(x, s1, b1, wqkv):
    def body(x_ref, s_ref, b_ref, w_ref, q_ref, k_ref, v_ref):
        h = _ln_in(x_ref[...], s_ref[...], b_ref[...])
        qkv = jnp.dot(h, w_ref[...], preferred_element_type=_F32)
        q_ref[...] = qkv[:, :D_A]
        k_ref[...] = qkv[:, D_A:2 * D_A]
        v_ref[...] = qkv[:, 2 * D_A:]

    tile = lambda i: (i, 0)
    full = lambda i: (0, 0)
    return pl.pallas_call(
        body,
        grid=(NT,),
        in_specs=[
            pl.BlockSpec((TT, D_A), tile),
            pl.BlockSpec((1, D_A), full),
            pl.BlockSpec((1, D_A), full),
            pl.BlockSpec((D_A, 3 * D_A), full),
        ],
        out_specs=[
            pl.BlockSpec((TT, D_A), tile),
            pl.BlockSpec((TT, D_A), tile),
            pl.BlockSpec((TT, D_A), tile),
        ],
        out_shape=[jax.ShapeDtypeStruct((T, D_A), _F32)] * 3,
    )(x, s1, b1, wqkv)


def _attn_call(q, k, v):
    def body(q_ref, k_ref, v_ref, o_ref):
        i = pl.program_id(0)

        def attn_len(L):
            row = i * TT + lax.broadcasted_iota(jnp.int32, (TT, L), 0)
            col = lax.broadcasted_iota(jnp.int32, (TT, L), 1)
            madd = jnp.where(col <= row, _F32(0.0), _F32(-1e9))
            outs = []
            for h in range(N_HEADS):
                hs = slice(h * D_H, (h + 1) * D_H)
                qh = q_ref[:, hs] * 0.125
                kh = k_ref[0:L, hs]
                vh = v_ref[0:L, hs]
                s = lax.dot_general(qh, kh, (((1,), (1,)), ((), ())),
                                    preferred_element_type=_F32) + madd
                m = jnp.max(s, axis=-1, keepdims=True)
                e = jnp.exp(s - m)
                rden = 1.0 / jnp.sum(e, axis=-1, keepdims=True)
                outs.append(jnp.dot(e, vh,
                                    preferred_element_type=_F32) * rden)
            o_ref[...] = jnp.concatenate(outs, axis=1)

        for pi in range(NT // 2):

            @pl.when(i // 2 == pi)
            def _(pi=pi):
                attn_len((pi + 1) * 2 * TT)

    return pl.pallas_call(
        body,
        grid=(NT,),
        in_specs=[
            pl.BlockSpec((TT, D_A), lambda i: (i, 0)),
            pl.BlockSpec((T, D_A), lambda i: (0, 0)),
            pl.BlockSpec((T, D_A), lambda i: (0, 0)),
        ],
        out_specs=pl.BlockSpec((TT, D_A), lambda i: (i, 0)),
        out_shape=jax.ShapeDtypeStruct((T, D_A), _F32),
    )(q, k, v)


def _woffn_call(x, attn, wo, s2, b2, w1, bb1, w2, bb2, want_zsum):
    def body(x_ref, a_ref, wo_ref, s_ref, b_ref, w1_ref, b1_ref,
             w2_ref, b2_ref, y_ref, *maybe_z):
        x1 = x_ref[...] + jnp.dot(a_ref[...], wo_ref[...],
                                  preferred_element_type=_F32)
        h2 = _ln_in(x1, s_ref[...], b_ref[...])
        ff = jax.nn.gelu(jnp.dot(h2, w1_ref[...],
                                 preferred_element_type=_F32) + b1_ref[...])
        y = x1 + jnp.dot(ff, w2_ref[...],
                         preferred_element_type=_F32) + b2_ref[...]
        y_ref[...] = y
        if maybe_z:
            maybe_z[0][...] = jnp.sum(y, axis=0, keepdims=True)[None]

    tile = lambda i: (i, 0)
    full = lambda i: (0, 0)
    out_specs = [pl.BlockSpec((TT, D_A), tile)]
    out_shape = [jax.ShapeDtypeStruct((T, D_A), _F32)]
    if want_zsum:
        out_specs.append(pl.BlockSpec((1, 1, D_A), lambda i: (i, 0, 0)))
        out_shape.append(jax.ShapeDtypeStruct((NT, 1, D_A), _F32))
    res = pl.pallas_call(
        body,
        grid=(NT,),
        in_specs=[
            pl.BlockSpec((TT, D_A), tile),
            pl.BlockSpec((TT, D_A), tile),
            pl.BlockSpec((D_A, D_A), full),
            pl.BlockSpec((1, D_A), full),
            pl.BlockSpec((1, D_A), full),
            pl.BlockSpec((D_A, D_FF), full),
            pl.BlockSpec((1, D_FF), full),
            pl.BlockSpec((D_FF, D_A), full),
            pl.BlockSpec((1, D_A), full),
        ],
        out_specs=out_specs,
        out_shape=out_shape,
    )(x, attn, wo, s2, b2, w1, bb1, w2, bb2)
    return res


def _route_m_call(zparts, wq2, wk2):
    def body(zp_ref, wq_ref, wk_ref, m_ref):
        z = jnp.sum(zp_ref[...][:, 0, :], axis=0, keepdims=True) * (1.0 / T)
        qf = jnp.dot(z, wq_ref[...], preferred_element_type=_F32)
        m = lax.dot_general(qf, wk_ref[...], (((1,), (1,)), ((), ())),
                            preferred_element_type=_F32)
        m_ref[...] = m * 0.125

    return pl.pallas_call(
        body,
        in_specs=[
            pl.BlockSpec((NT, 1, D_A), lambda: (0, 0, 0)),
            pl.BlockSpec((D_A, N_ASPECTS * D_K), lambda: (0, 0)),
            pl.BlockSpec((D_POOL, N_ASPECTS * D_K), lambda: (0, 0)),
        ],
        out_specs=pl.BlockSpec((1, D_POOL), lambda: (0, 0)),
        out_shape=jax.ShapeDtypeStruct((1, D_POOL), _F32),
    )(zparts, wq2, wk2)


def _combined_call(pool, m):
    def body(p_ref, m_ref, o_ref):
        o_ref[...] = lax.dot_general(
            p_ref[...], m_ref[...], (((1,), (1,)), ((), ())),
            preferred_element_type=_F32)

    return pl.pallas_call(
        body,
        grid=(NPT,),
        in_specs=[
            pl.BlockSpec((PT, D_POOL), lambda i: (i, 0)),
            pl.BlockSpec((1, D_POOL), lambda i: (0, 0)),
        ],
        out_specs=pl.BlockSpec((PT, 1), lambda i: (i, 0)),
        out_shape=jax.ShapeDtypeStruct((N_POOL, 1), _F32),
    )(pool, m)


def _topk_call(c2d, lam, warm):
    rows, cols = c2d.shape

    def body(c_ref, lam_ref, warm_ref, a_ref, i_ref):
        c = c_ref[...] * lam_ref[0]
        cmax = jnp.max(c)
        e = jnp.exp(c - cmax)
        soft = e / jnp.sum(e)
        flat = (lax.broadcasted_iota(jnp.int32, (rows, cols), 0) * cols
                + lax.broadcasted_iota(jnp.int32, (rows, cols), 1))
        cur = soft
        vals = []
        for kk in range(TOP_K):
            mx = jnp.max(cur)
            am = jnp.min(jnp.where(cur == mx, flat, jnp.int32(N_POOL)))
            vals.append(mx)
            i_ref[kk] = am
            cur = jnp.where(flat == am, _F32(-1.0), cur)
        vsum = vals[0]
        for kk in range(1, TOP_K):
            vsum = vsum + vals[kk]
        warmb = warm_ref[0] != 0
        for kk in range(TOP_K):
            a_ref[kk] = jnp.where(warmb, vals[kk],
                                  vals[kk] / (vsum + 1e-9))

    return pl.pallas_call(
        body,
        in_specs=[
            pl.BlockSpec((rows, cols), lambda: (0, 0)),
            pl.BlockSpec(memory_space=pltpu.SMEM),
            pl.BlockSpec(memory_space=pltpu.SMEM),
        ],
        out_specs=[
            pl.BlockSpec(memory_space=pltpu.SMEM),
            pl.BlockSpec(memory_space=pltpu.SMEM),
        ],
        out_shape=[
            jax.ShapeDtypeStruct((TOP_K,), _F32),
            jax.ShapeDtypeStruct((TOP_K,), jnp.int32),
        ],
    )(c2d, lam, warm)


def _wm_call(au, bv, alpha16, w_base, gamma):
    def body(au_ref, bv_ref, al_ref, wb_ref, g_ref, o_ref):
        delta = jnp.dot(au_ref[...] * al_ref[...], bv_ref[...],
                        preferred_element_type=_F32)
        o_ref[...] = wb_ref[...] + g_ref[0] * delta

    return pl.pallas_call(
        body,
        in_specs=[
            pl.BlockSpec((D_B, 2 * TOP_K), lambda: (0, 0)),
            pl.BlockSpec((2 * TOP_K, D_A), lambda: (0, 0)),
            pl.BlockSpec((1, 2 * TOP_K), lambda: (0, 0)),
            pl.BlockSpec((D_B, D_A), lambda: (0, 0)),
            pl.BlockSpec(memory_space=pltpu.SMEM),
        ],
        out_specs=pl.BlockSpec((D_B, D_A), lambda: (0, 0)),
        out_shape=jax.ShapeDtypeStruct((D_B, D_A), _F32),
    )(au, bv, alpha16, w_base, gamma)


def _hmid_call(h_a, wm, b_base, s, b):
    def body(x_ref, w_ref, bb_ref, s_ref, b_ref, o_ref):
        t = lax.dot_general(x_ref[...], w_ref[...],
                            (((1,), (1,)), ((), ())),
                            preferred_element_type=_F32) + bb_ref[...]
        o_ref[...] = _ln_in(t, s_ref[...], b_ref[...])

    tile = lambda i: (i, 0)
    full = lambda i: (0, 0)
    return pl.pallas_call(
        body,
        grid=(NT,),
        in_specs=[
            pl.BlockSpec((TT, D_A), tile),
            pl.BlockSpec((D_B, D_A), full),
            pl.BlockSpec((1, D_B), full),
            pl.BlockSpec((1, D_B), full),
            pl.BlockSpec((1, D_B), full),
        ],
        out_specs=pl.BlockSpec((TT, D_B), tile),
        out_shape=jax.ShapeDtypeStruct((T, D_B), _F32),
    )(h_a, wm, b_base, s, b)


def _lmhead_call(x, w):
    def body(x_ref, w_ref, o_ref):
        o_ref[...] = jnp.dot(x_ref[...], w_ref[...],
                             preferred_element_type=_F32)

    return pl.pallas_call(
        body,
        grid=(NVT,),
        in_specs=[
            pl.BlockSpec((T, D_B), lambda j: (0, 0)),
            pl.BlockSpec((D_B, VT), lambda j: (0, j)),
        ],
        out_specs=pl.BlockSpec((T, VT), lambda j: (0, j)),
        out_shape=jax.ShapeDtypeStruct((T, VOCAB), _F32),
    )(x, w)


def kernel(input_ids, lambda_val, is_warmup, embed_table, a_ln1_s, a_ln1_b,
           a_wqkv, a_wo, a_ln2_s, a_ln2_b, a_w1, a_b1, a_w2, a_b2,
           pool_vectors, w_key, w_query, w_base, b_base, gamma, asm_ln_s,
           asm_ln_b, b_ln1_s, b_ln1_b, b_wqkv, b_wo, b_ln2_s, b_ln2_b,
           b_w1, b_b1, b_w2, b_b2, lm_head_w):
    row2 = lambda a: jnp.asarray(a, _F32).reshape(1, -1)

    ids = input_ids.reshape(T).astype(jnp.int32)
    g = _embed_gather(embed_table, ids)
    pos = _pos_enc_const(T, D_A)

    q, k, v, x = _qkv_a_call(g, pos, row2(a_ln1_s), row2(a_ln1_b), a_wqkv)
    attn = _attn_call(q, k, v)
    h_a, zparts = _woffn_call(x, attn, a_wo, row2(a_ln2_s), row2(a_ln2_b),
                              a_w1, row2(a_b1), a_w2, row2(a_b2),
                              want_zsum=True)

    wq2 = w_query.transpose(1, 0, 2).reshape(D_A, N_ASPECTS * D_K)
    wk2 = w_key.transpose(1, 0, 2).reshape(D_POOL, N_ASPECTS * D_K)
    m = _route_m_call(zparts, wq2, wk2)
    combined = _combined_call(pool_vectors, m)
    lam = jnp.asarray(lambda_val, _F32).reshape(1)
    warm = jnp.asarray(is_warmup, jnp.int32).reshape(1)
    alphas, indices = _topk_call(combined.reshape(64, 128), lam, warm)

    gathered = _pool_gather(pool_vectors, indices)
    au = gathered[:, :D_B * R].reshape(TOP_K, D_B, R).transpose(1, 0, 2)
    au = au.reshape(D_B, TOP_K * R)
    bv = gathered[:, D_B * R:].reshape(TOP_K * R, D_A)
    alpha16 = jnp.repeat(alphas, R).reshape(1, TOP_K * R)
    wm = _wm_call(au, bv, alpha16, w_base, gamma.reshape(1))

    h_mid = _hmid_call(h_a, wm, row2(b_base), row2(asm_ln_s), row2(asm_ln_b))

    q2, k2, v2 = _qkv_b_call(h_mid, row2(b_ln1_s), row2(b_ln1_b), b_wqkv)
    attn2 = _attn_call(q2, k2, v2)
    [h_out] = _woffn_call(h_mid, attn2, b_wo, row2(b_ln2_s), row2(b_ln2_b),
                          b_w1, row2(b_b1), b_w2, row2(b_b2),
                          want_zsum=False)

    logits = _lmhead_call(h_out, lm_head_w)
    return logits.reshape(1, T, VOCAB)


# speedup vs baseline: 2.3079x; 1.0650x over previous
import functools

import jax
import jax.numpy as jnp
from jax import lax
from jax.experimental import pallas as pl
from jax.experimental.pallas import tpu as pltpu
from jax.experimental.pallas import tpu_sc as plsc

VOCAB = 32000
D_A = 768
D_B = 768
N_HEADS = 12
D_H = 64
D_FF = 3072
N_POOL = 8192
R = 2
TOP_K = 8
D_K = 64
N_ASPECTS = 4
T = 2048
D_POOL = R * (D_A + D_B)

TT = 256
NT = T // TT
PT = 1024
NPT = N_POOL // PT
VT = 1280
NVT = VOCAB // VT

_F32 = jnp.float32


def _ln_in(x, s, b):
    m = jnp.mean(x, axis=-1, keepdims=True)
    v = jnp.mean((x - m) ** 2, axis=-1, keepdims=True)
    return (x - m) * lax.rsqrt(v + 1e-5) * s + b


def _pos_enc_const(seq_len, d_model):
    pos = jnp.arange(seq_len)[:, None]
    i = jnp.arange(d_model // 2)[None, :]
    angle = pos / 10000 ** (2 * i / d_model)
    enc = jnp.concatenate([jnp.sin(angle), jnp.cos(angle)], axis=-1)
    return enc[:, :d_model].astype(_F32)


def _embed_gather(table, idx):
    info = plsc.get_sparse_core_info()
    nc, ns = info.num_cores, info.num_subcores
    nw = nc * ns
    bpw = T // nw
    mesh = plsc.VectorSubcoreMesh(core_axis_name="c", subcore_axis_name="s")

    @functools.partial(
        pl.kernel, mesh=mesh,
        out_type=jax.ShapeDtypeStruct((T, D_A), _F32),
        scratch_types=[
            pltpu.VMEM((bpw,), jnp.int32),
            pltpu.VMEM((bpw, D_A), _F32),
            pltpu.SemaphoreType.DMA,
        ],
    )
    def k(table_hbm, idx_hbm, out_hbm, idx_v, rows_v, sem):
        wid = lax.axis_index("s") * nc + lax.axis_index("c")
        base = wid * bpw
        pltpu.sync_copy(idx_hbm.at[pl.ds(base, bpw)], idx_v)
        pltpu.async_copy(table_hbm.at[idx_v], rows_v, sem).wait()
        pltpu.sync_copy(rows_v, out_hbm.at[pl.ds(base, bpw)])

    return k(table, idx)


def _pool_gather(pool, idx):
    info = plsc.get_sparse_core_info()
    nc = info.num_cores
    mesh = plsc.VectorSubcoreMesh(core_axis_name="c", subcore_axis_name="s")

    @functools.partial(
        pl.kernel, mesh=mesh,
        out_type=jax.ShapeDtypeStruct((TOP_K, D_POOL), _F32),
        scratch_types=[
            pltpu.VMEM((TOP_K,), jnp.int32),
            pltpu.VMEM((TOP_K, D_POOL), _F32),
            pltpu.SemaphoreType.DMA,
        ],
    )
    def k(pool_hbm, idx_hbm, out_hbm, idx_v, rows_v, sem):
        wid = lax.axis_index("s") * nc + lax.axis_index("c")

        @pl.when(wid == 0)
        def _():
            pltpu.sync_copy(idx_hbm, idx_v)
            pltpu.async_copy(pool_hbm.at[idx_v], rows_v, sem).wait()
            pltpu.sync_copy(rows_v, out_hbm)

    return k(pool, idx)


DP = 128
DAP = N_HEADS * DP


def _scatter_heads(qkv, off, ref):
    for h in range(N_HEADS):
        ref[:, h * DP:h * DP + D_H] = qkv[:, off + h * D_H:off + (h + 1) * D_H]
        ref[:, h * DP + D_H:(h + 1) * DP] = jnp.zeros((qkv.shape[0], D_H),
                                                      _F32)


def _qkv_a_call(g, pos, s1, b1, wqkv):
    def body(g_ref, p_ref, s_ref, b_ref, w_ref, q_ref, k_ref, v_ref, x_ref):
        x = g_ref[...] + p_ref[...]
        x_ref[...] = x
        h = _ln_in(x, s_ref[...], b_ref[...])
        qkv = jnp.dot(h, w_ref[...], preferred_element_type=_F32)
        _scatter_heads(qkv, 0, q_ref)
        _scatter_heads(qkv, D_A, k_ref)
        _scatter_heads(qkv, 2 * D_A, v_ref)

    tile = lambda i: (i, 0)
    full = lambda i: (0, 0)
    return pl.pallas_call(
        body,
        grid=(NT,),
        in_specs=[
            pl.BlockSpec((TT, D_A), tile),
            pl.BlockSpec((TT, D_A), tile),
            pl.BlockSpec((1, D_A), full),
            pl.BlockSpec((1, D_A), full),
            pl.BlockSpec((D_A, 3 * D_A), full),
        ],
        out_specs=[
            pl.BlockSpec((TT, DAP), tile),
            pl.BlockSpec((TT, DAP), tile),
            pl.BlockSpec((TT, DAP), tile),
            pl.BlockSpec((TT, D_A), tile),
        ],
        out_shape=[jax.ShapeDtypeStruct((T, DAP), _F32)] * 3
        + [jax.ShapeDtypeStruct((T, D_A), _F32)],
    )(g, pos, s1, b1, wqkv)


def _qkv_b_call(x, s1, b1, wqkv):
    def body(x_ref, s_ref, b_ref, w_ref, q_ref, k_ref, v_ref):
        h = _ln_in(x_ref[...], s_ref[...], b_ref[...])
        qkv = jnp.dot(h, w_ref[...], preferred_element_type=_F32)
        _scatter_heads(qkv, 0, q_ref)
        _scatter_heads(qkv, D_A, k_ref)
        _scatter_heads(qkv, 2 * D_A, v_ref)

    tile = lambda i: (i, 0)
    full = lambda i: (0, 0)
    return pl.pallas_call(
        body,
        grid=(NT,),
        in_specs=[
            pl.BlockSpec((TT, D_A), tile),
            pl.BlockSpec((1, D_A), full),
            pl.BlockSpec((1, D_A), full),
            pl.BlockSpec((D_A, 3 * D_A), full),
        ],
        out_specs=[
            pl.BlockSpec((TT, DAP), tile),
            pl.BlockSpec((TT, DAP), tile),
            pl.BlockSpec((TT, DAP), tile),
        ],
        out_shape=[jax.ShapeDtypeStruct((T, DAP), _F32)] * 3,
    )(x, s1, b1, wqkv)


def _attn_call(q, k, v):
    def body(q_ref, k_ref, v_ref, o_ref):
        i = pl.program_id(0)

        def attn_len(L):
            row = i * TT + lax.broadcasted_iota(jnp.int32, (TT, L), 0)
            col = lax.broadcasted_iota(jnp.int32, (TT, L), 1)
            madd = jnp.where(col <= row, _F32(0.0), _F32(-1e9))
            outs = []
            for h in range(N_HEADS):
                hs = slice(h * DP, (h + 1) * DP)
                qh = q_ref[:, hs] * 0.125
                kh = k_ref[0:L, hs]
                vh = v_ref[0:L, hs]
                s = lax.dot_general(qh, kh, (((1,), (1,)), ((), ())),
                                    preferred_element_type=_F32) + madd
                m = jnp.max(s, axis=-1, keepdims=True)
                e = jnp.exp(s - m)
                rden = 1.0 / jnp.sum(e, axis=-1, keepdims=True)
                outs.append(jnp.dot(e, vh,
                                    preferred_element_type=_F32)[:, :D_H]
                            * rden)
            o_ref[...] = jnp.concatenate(outs, axis=1)

        for pi in range(NT // 2):

            @pl.when(i // 2 == pi)
            def _(pi=pi):
                attn_len((pi + 1) * 2 * TT)

    return pl.pallas_call(
        body,
        grid=(NT,),
        in_specs=[
            pl.BlockSpec((TT, DAP), lambda i: (i, 0)),
            pl.BlockSpec((T, DAP), lambda i: (0, 0)),
            pl.BlockSpec((T, DAP), lambda i: (0, 0)),
        ],
        out_specs=pl.BlockSpec((TT, D_A), lambda i: (i, 0)),
        out_shape=jax.ShapeDtypeStruct((T, D_A), _F32),
    )(q, k, v)


def _woffn_call(x, attn, wo, s2, b2, w1, bb1, w2, bb2, want_zsum):
    def body(x_ref, a_ref, wo_ref, s_ref, b_ref, w1_ref, b1_ref,
             w2_ref, b2_ref, y_ref, *maybe_z):
        x1 = x_ref[...] + jnp.dot(a_ref[...], wo_ref[...],
                                  preferred_element_type=_F32)
        h2 = _ln_in(x1, s_ref[...], b_ref[...])
        ff = jax.nn.gelu(jnp.dot(h2, w1_ref[...],
                                 preferred_element_type=_F32) + b1_ref[...])
        y = x1 + jnp.dot(ff, w2_ref[...],
                         preferred_element_type=_F32) + b2_ref[...]
        y_ref[...] = y
        if maybe_z:
            maybe_z[0][...] = jnp.sum(y, axis=0, keepdims=True)[None]

    tile = lambda i: (i, 0)
    full = lambda i: (0, 0)
    out_specs = [pl.BlockSpec((TT, D_A), tile)]
    out_shape = [jax.ShapeDtypeStruct((T, D_A), _F32)]
    if want_zsum:
        out_specs.append(pl.BlockSpec((1, 1, D_A), lambda i: (i, 0, 0)))
        out_shape.append(jax.ShapeDtypeStruct((NT, 1, D_A), _F32))
    res = pl.pallas_call(
        body,
        grid=(NT,),
        in_specs=[
            pl.BlockSpec((TT, D_A), tile),
            pl.BlockSpec((TT, D_A), tile),
            pl.BlockSpec((D_A, D_A), full),
            pl.BlockSpec((1, D_A), full),
            pl.BlockSpec((1, D_A), full),
            pl.BlockSpec((D_A, D_FF), full),
            pl.BlockSpec((1, D_FF), full),
            pl.BlockSpec((D_FF, D_A), full),
            pl.BlockSpec((1, D_A), full),
        ],
        out_specs=out_specs,
        out_shape=out_shape,
    )(x, attn, wo, s2, b2, w1, bb1, w2, bb2)
    return res


def _route_m_call(zparts, wq2, wk2):
    def body(zp_ref, wq_ref, wk_ref, m_ref):
        z = jnp.sum(zp_ref[...][:, 0, :], axis=0, keepdims=True) * (1.0 / T)
        qf = jnp.dot(z, wq_ref[...], preferred_element_type=_F32)
        m = lax.dot_general(qf, wk_ref[...], (((1,), (1,)), ((), ())),
                            preferred_element_type=_F32)
        m_ref[...] = m * 0.125

    return pl.pallas_call(
        body,
        in_specs=[
            pl.BlockSpec((NT, 1, D_A), lambda: (0, 0, 0)),
            pl.BlockSpec((D_A, N_ASPECTS * D_K), lambda: (0, 0)),
            pl.BlockSpec((D_POOL, N_ASPECTS * D_K), lambda: (0, 0)),
        ],
        out_specs=pl.BlockSpec((1, D_POOL), lambda: (0, 0)),
        out_shape=jax.ShapeDtypeStruct((1, D_POOL), _F32),
    )(zparts, wq2, wk2)


def _combined_call(pool, m):
    def body(p_ref, m_ref, o_ref):
        o_ref[...] = lax.dot_general(
            p_ref[...], m_ref[...], (((1,), (1,)), ((), ())),
            preferred_element_type=_F32)

    return pl.pallas_call(
        body,
        grid=(NPT,),
        in_specs=[
            pl.BlockSpec((PT, D_POOL), lambda i: (i, 0)),
            pl.BlockSpec((1, D_POOL), lambda i: (0, 0)),
        ],
        out_specs=pl.BlockSpec((PT, 1), lambda i: (i, 0)),
        out_shape=jax.ShapeDtypeStruct((N_POOL, 1), _F32),
    )(pool, m)


def _topk_call(c2d, lam, warm):
    rows, cols = c2d.shape

    def body(c_ref, lam_ref, warm_ref, a_ref, i_ref):
        c = c_ref[...] * lam_ref[0]
        cmax = jnp.max(c)
        e = jnp.exp(c - cmax)
        soft = e / jnp.sum(e)
        flat = (lax.broadcasted_iota(jnp.int32, (rows, cols), 0) * cols
                + lax.broadcasted_iota(jnp.int32, (rows, cols), 1))
        cur = soft
        vals = []
        for kk in range(TOP_K):
            mx = jnp.max(cur)
            am = jnp.min(jnp.where(cur == mx, flat, jnp.int32(N_POOL)))
            vals.append(mx)
            i_ref[kk] = am
            cur = jnp.where(flat == am, _F32(-1.0), cur)
        vsum = vals[0]
        for kk in range(1, TOP_K):
            vsum = vsum + vals[kk]
        warmb = warm_ref[0] != 0
        for kk in range(TOP_K):
            a_ref[kk] = jnp.where(warmb, vals[kk],
                                  vals[kk] / (vsum + 1e-9))

    return pl.pallas_call(
        body,
        in_specs=[
            pl.BlockSpec((rows, cols), lambda: (0, 0)),
            pl.BlockSpec(memory_space=pltpu.SMEM),
            pl.BlockSpec(memory_space=pltpu.SMEM),
        ],
        out_specs=[
            pl.BlockSpec(memory_space=pltpu.SMEM),
            pl.BlockSpec(memory_space=pltpu.SMEM),
        ],
        out_shape=[
            jax.ShapeDtypeStruct((TOP_K,), _F32),
            jax.ShapeDtypeStruct((TOP_K,), jnp.int32),
        ],
    )(c2d, lam, warm)


def _wm_call(au, bv, alpha16, w_base, gamma):
    def body(au_ref, bv_ref, al_ref, wb_ref, g_ref, o_ref):
        delta = jnp.dot(au_ref[...] * al_ref[...], bv_ref[...],
                        preferred_element_type=_F32)
        o_ref[...] = wb_ref[...] + g_ref[0] * delta

    return pl.pallas_call(
        body,
        in_specs=[
            pl.BlockSpec((D_B, 2 * TOP_K), lambda: (0, 0)),
            pl.BlockSpec((2 * TOP_K, D_A), lambda: (0, 0)),
            pl.BlockSpec((1, 2 * TOP_K), lambda: (0, 0)),
            pl.BlockSpec((D_B, D_A), lambda: (0, 0)),
            pl.BlockSpec(memory_space=pltpu.SMEM),
        ],
        out_specs=pl.BlockSpec((D_B, D_A), lambda: (0, 0)),
        out_shape=jax.ShapeDtypeStruct((D_B, D_A), _F32),
    )(au, bv, alpha16, w_base, gamma)


def _hmid_call(h_a, wm, b_base, s, b):
    def body(x_ref, w_ref, bb_ref, s_ref, b_ref, o_ref):
        t = lax.dot_general(x_ref[...], w_ref[...],
                            (((1,), (1,)), ((), ())),
                            preferred_element_type=_F32) + bb_ref[...]
        o_ref[...] = _ln_in(t, s_ref[...], b_ref[...])

    tile = lambda i: (i, 0)
    full = lambda i: (0, 0)
    return pl.pallas_call(
        body,
        grid=(NT,),
        in_specs=[
            pl.BlockSpec((TT, D_A), tile),
            pl.BlockSpec((D_B, D_A), full),
            pl.BlockSpec((1, D_B), full),
            pl.BlockSpec((1, D_B), full),
            pl.BlockSpec((1, D_B), full),
        ],
        out_specs=pl.BlockSpec((TT, D_B), tile),
        out_shape=jax.ShapeDtypeStruct((T, D_B), _F32),
    )(h_a, wm, b_base, s, b)


def _lmhead_call(x, w):
    def body(x_ref, w_ref, o_ref):
        o_ref[...] = jnp.dot(x_ref[...], w_ref[...],
                             preferred_element_type=_F32)

    return pl.pallas_call(
        body,
        grid=(NVT,),
        in_specs=[
            pl.BlockSpec((T, D_B), lambda j: (0, 0)),
            pl.BlockSpec((D_B, VT), lambda j: (0, j)),
        ],
        out_specs=pl.BlockSpec((T, VT), lambda j: (0, j)),
        out_shape=jax.ShapeDtypeStruct((T, VOCAB), _F32),
    )(x, w)


def kernel(input_ids, lambda_val, is_warmup, embed_table, a_ln1_s, a_ln1_b,
           a_wqkv, a_wo, a_ln2_s, a_ln2_b, a_w1, a_b1, a_w2, a_b2,
           pool_vectors, w_key, w_query, w_base, b_base, gamma, asm_ln_s,
           asm_ln_b, b_ln1_s, b_ln1_b, b_wqkv, b_wo, b_ln2_s, b_ln2_b,
           b_w1, b_b1, b_w2, b_b2, lm_head_w):
    row2 = lambda a: jnp.asarray(a, _F32).reshape(1, -1)

    ids = input_ids.reshape(T).astype(jnp.int32)
    g = _embed_gather(embed_table, ids)
    pos = _pos_enc_const(T, D_A)

    q, k, v, x = _qkv_a_call(g, pos, row2(a_ln1_s), row2(a_ln1_b), a_wqkv)
    attn = _attn_call(q, k, v)
    h_a, zparts = _woffn_call(x, attn, a_wo, row2(a_ln2_s), row2(a_ln2_b),
                              a_w1, row2(a_b1), a_w2, row2(a_b2),
                              want_zsum=True)

    wq2 = w_query.transpose(1, 0, 2).reshape(D_A, N_ASPECTS * D_K)
    wk2 = w_key.transpose(1, 0, 2).reshape(D_POOL, N_ASPECTS * D_K)
    m = _route_m_call(zparts, wq2, wk2)
    combined = _combined_call(pool_vectors, m)
    lam = jnp.asarray(lambda_val, _F32).reshape(1)
    warm = jnp.asarray(is_warmup, jnp.int32).reshape(1)
    alphas, indices = _topk_call(combined.reshape(64, 128), lam, warm)

    gathered = _pool_gather(pool_vectors, indices)
    au = gathered[:, :D_B * R].reshape(TOP_K, D_B, R).transpose(1, 0, 2)
    au = au.reshape(D_B, TOP_K * R)
    bv = gathered[:, D_B * R:].reshape(TOP_K * R, D_A)
    alpha16 = jnp.repeat(alphas, R).reshape(1, TOP_K * R)
    wm = _wm_call(au, bv, alpha16, w_base, gamma.reshape(1))

    h_mid = _hmid_call(h_a, wm, row2(b_base), row2(asm_ln_s), row2(asm_ln_b))

    q2, k2, v2 = _qkv_b_call(h_mid, row2(b_ln1_s), row2(b_ln1_b), b_wqkv)
    attn2 = _attn_call(q2, k2, v2)
    [h_out] = _woffn_call(h_mid, attn2, b_wo, row2(b_ln2_s), row2(b_ln2_b),
                          b_w1, row2(b_b1), b_w2, row2(b_b2),
                          want_zsum=False)

    logits = _lmhead_call(h_out, lm_head_w)
    return logits.reshape(1, T, VOCAB)


# speedup vs baseline: 2.4577x; 1.0649x over previous
import functools

import jax
import jax.numpy as jnp
from jax import lax
from jax.experimental import pallas as pl
from jax.experimental.pallas import tpu as pltpu
from jax.experimental.pallas import tpu_sc as plsc

VOCAB = 32000
D_A = 768
D_B = 768
N_HEADS = 12
D_H = 64
D_FF = 3072
N_POOL = 8192
R = 2
TOP_K = 8
D_K = 64
N_ASPECTS = 4
T = 2048
D_POOL = R * (D_A + D_B)

TT = 256
NT = T // TT
PT = 1024
NPT = N_POOL // PT
VT = 1280
NVT = VOCAB // VT

_F32 = jnp.float32


def _ln_in(x, s, b):
    m = jnp.mean(x, axis=-1, keepdims=True)
    v = jnp.mean((x - m) ** 2, axis=-1, keepdims=True)
    return (x - m) * lax.rsqrt(v + 1e-5) * s + b


def _pos_enc_const(seq_len, d_model):
    pos = jnp.arange(seq_len)[:, None]
    i = jnp.arange(d_model // 2)[None, :]
    angle = pos / 10000 ** (2 * i / d_model)
    enc = jnp.concatenate([jnp.sin(angle), jnp.cos(angle)], axis=-1)
    return enc[:, :d_model].astype(_F32)


def _embed_gather(table, idx):
    info = plsc.get_sparse_core_info()
    nc, ns = info.num_cores, info.num_subcores
    nw = nc * ns
    bpw = T // nw
    mesh = plsc.VectorSubcoreMesh(core_axis_name="c", subcore_axis_name="s")

    @functools.partial(
        pl.kernel, mesh=mesh,
        out_type=jax.ShapeDtypeStruct((T, D_A), _F32),
        scratch_types=[
            pltpu.VMEM((bpw,), jnp.int32),
            pltpu.VMEM((bpw, D_A), _F32),
            pltpu.SemaphoreType.DMA,
        ],
    )
    def k(table_hbm, idx_hbm, out_hbm, idx_v, rows_v, sem):
        wid = lax.axis_index("s") * nc + lax.axis_index("c")
        base = wid * bpw
        pltpu.sync_copy(idx_hbm.at[pl.ds(base, bpw)], idx_v)
        pltpu.async_copy(table_hbm.at[idx_v], rows_v, sem).wait()
        pltpu.sync_copy(rows_v, out_hbm.at[pl.ds(base, bpw)])

    return k(table, idx)


def _pool_gather(pool, idx):
    info = plsc.get_sparse_core_info()
    nc = info.num_cores
    mesh = plsc.VectorSubcoreMesh(core_axis_name="c", subcore_axis_name="s")

    @functools.partial(
        pl.kernel, mesh=mesh,
        out_type=jax.ShapeDtypeStruct((TOP_K, D_POOL), _F32),
        scratch_types=[
            pltpu.VMEM((TOP_K,), jnp.int32),
            pltpu.VMEM((TOP_K, D_POOL), _F32),
            pltpu.SemaphoreType.DMA,
        ],
    )
    def k(pool_hbm, idx_hbm, out_hbm, idx_v, rows_v, sem):
        wid = lax.axis_index("s") * nc + lax.axis_index("c")

        @pl.when(wid == 0)
        def _():
            pltpu.sync_copy(idx_hbm, idx_v)
            pltpu.async_copy(pool_hbm.at[idx_v], rows_v, sem).wait()
            pltpu.sync_copy(rows_v, out_hbm)

    return k(pool, idx)


DP = 128
DAP = N_HEADS * DP


def _pad_heads(qkv, off):
    pieces = []
    for h in range(N_HEADS):
        pieces.append(qkv[:, off + h * D_H:off + (h + 1) * D_H])
        pieces.append(jnp.zeros((qkv.shape[0], DP - D_H), _F32))
    return jnp.concatenate(pieces, axis=1)


def _attn_inner(i, qkv, k_scr, v_scr, o_ref):
    def attn_len(L):
        row = i * TT + lax.broadcasted_iota(jnp.int32, (TT, L), 0)
        col = lax.broadcasted_iota(jnp.int32, (TT, L), 1)
        madd = jnp.where(col <= row, _F32(0.0), _F32(-1e9))
        outs = []
        for h in range(N_HEADS):
            qh = qkv[:, h * D_H:(h + 1) * D_H] * 0.125
            kh = k_scr[0:L, h * DP:(h + 1) * DP]
            vh = v_scr[0:L, h * DP:(h + 1) * DP]
            s = lax.dot_general(
                jnp.concatenate(
                    [qh, jnp.zeros((TT, DP - D_H), _F32)], axis=1),
                kh, (((1,), (1,)), ((), ())),
                preferred_element_type=_F32) + madd
            m = jnp.max(s, axis=-1, keepdims=True)
            e = jnp.exp(s - m)
            rden = 1.0 / jnp.sum(e, axis=-1, keepdims=True)
            outs.append(jnp.dot(e, vh,
                                preferred_element_type=_F32)[:, :D_H] * rden)
        o_ref[...] = jnp.concatenate(outs, axis=1)

    for pi in range(NT // 2):

        @pl.when(i // 2 == pi)
        def _(pi=pi):
            attn_len((pi + 1) * 2 * TT)


def _qkvattn_a_call(g, pos, s1, b1, wqkv):
    def body(g_ref, p_ref, s_ref, b_ref, w_ref, a_ref, x_ref, k_scr, v_scr):
        i = pl.program_id(0)
        x = g_ref[...] + p_ref[...]
        x_ref[...] = x
        h = _ln_in(x, s_ref[...], b_ref[...])
        qkv = jnp.dot(h, w_ref[...], preferred_element_type=_F32)
        k_scr[pl.ds(i * TT, TT), :] = _pad_heads(qkv, D_A)
        v_scr[pl.ds(i * TT, TT), :] = _pad_heads(qkv, 2 * D_A)

        @pl.when(i % 2 == 0)
        def _():
            k_scr[pl.ds((i + 1) * TT, TT), :] = jnp.zeros((TT, DAP), _F32)
            v_scr[pl.ds((i + 1) * TT, TT), :] = jnp.zeros((TT, DAP), _F32)

        _attn_inner(i, qkv, k_scr, v_scr, a_ref)

    tile = lambda i: (i, 0)
    full = lambda i: (0, 0)
    return pl.pallas_call(
        body,
        grid=(NT,),
        in_specs=[
            pl.BlockSpec((TT, D_A), tile),
            pl.BlockSpec((TT, D_A), tile),
            pl.BlockSpec((1, D_A), full),
            pl.BlockSpec((1, D_A), full),
            pl.BlockSpec((D_A, 3 * D_A), full),
        ],
        out_specs=[
            pl.BlockSpec((TT, D_A), tile),
            pl.BlockSpec((TT, D_A), tile),
        ],
        out_shape=[jax.ShapeDtypeStruct((T, D_A), _F32)] * 2,
        scratch_shapes=[
            pltpu.VMEM((T, DAP), _F32),
            pltpu.VMEM((T, DAP), _F32),
        ],
    )(g, pos, s1, b1, wqkv)


def _qkvattn_b_call(x, s1, b1, wqkv):
    def body(x_ref, s_ref, b_ref, w_ref, a_ref, k_scr, v_scr):
        i = pl.program_id(0)
        h = _ln_in(x_ref[...], s_ref[...], b_ref[...])
        qkv = jnp.dot(h, w_ref[...], preferred_element_type=_F32)
        k_scr[pl.ds(i * TT, TT), :] = _pad_heads(qkv, D_A)
        v_scr[pl.ds(i * TT, TT), :] = _pad_heads(qkv, 2 * D_A)

        @pl.when(i % 2 == 0)
        def _():
            k_scr[pl.ds((i + 1) * TT, TT), :] = jnp.zeros((TT, DAP), _F32)
            v_scr[pl.ds((i + 1) * TT, TT), :] = jnp.zeros((TT, DAP), _F32)

        _attn_inner(i, qkv, k_scr, v_scr, a_ref)

    tile = lambda i: (i, 0)
    full = lambda i: (0, 0)
    return pl.pallas_call(
        body,
        grid=(NT,),
        in_specs=[
            pl.BlockSpec((TT, D_A), tile),
            pl.BlockSpec((1, D_A), full),
            pl.BlockSpec((1, D_A), full),
            pl.BlockSpec((D_A, 3 * D_A), full),
        ],
        out_specs=pl.BlockSpec((TT, D_A), tile),
        out_shape=jax.ShapeDtypeStruct((T, D_A), _F32),
        scratch_shapes=[
            pltpu.VMEM((T, DAP), _F32),
            pltpu.VMEM((T, DAP), _F32),
        ],
    )(x, s1, b1, wqkv)


def _woffn_call(x, attn, wo, s2, b2, w1, bb1, w2, bb2, want_zsum):
    def body(x_ref, a_ref, wo_ref, s_ref, b_ref, w1_ref, b1_ref,
             w2_ref, b2_ref, y_ref, *maybe_z):
        x1 = x_ref[...] + jnp.dot(a_ref[...], wo_ref[...],
                                  preferred_element_type=_F32)
        h2 = _ln_in(x1, s_ref[...], b_ref[...])
        ff = jax.nn.gelu(jnp.dot(h2, w1_ref[...],
                                 preferred_element_type=_F32) + b1_ref[...])
        y = x1 + jnp.dot(ff, w2_ref[...],
                         preferred_element_type=_F32) + b2_ref[...]
        y_ref[...] = y
        if maybe_z:
            maybe_z[0][...] = jnp.sum(y, axis=0, keepdims=True)[None]

    tile = lambda i: (i, 0)
    full = lambda i: (0, 0)
    out_specs = [pl.BlockSpec((TT, D_A), tile)]
    out_shape = [jax.ShapeDtypeStruct((T, D_A), _F32)]
    if want_zsum:
        out_specs.append(pl.BlockSpec((1, 1, D_A), lambda i: (i, 0, 0)))
        out_shape.append(jax.ShapeDtypeStruct((NT, 1, D_A), _F32))
    res = pl.pallas_call(
        body,
        grid=(NT,),
        in_specs=[
            pl.BlockSpec((TT, D_A), tile),
            pl.BlockSpec((TT, D_A), tile),
            pl.BlockSpec((D_A, D_A), full),
            pl.BlockSpec((1, D_A), full),
            pl.BlockSpec((1, D_A), full),
            pl.BlockSpec((D_A, D_FF), full),
            pl.BlockSpec((1, D_FF), full),
            pl.BlockSpec((D_FF, D_A), full),
            pl.BlockSpec((1, D_A), full),
        ],
        out_specs=out_specs,
        out_shape=out_shape,
    )(x, attn, wo, s2, b2, w1, bb1, w2, bb2)
    return res


def _route_m_call(zparts, wq2, wk2):
    def body(zp_ref, wq_ref, wk_ref, m_ref):
        z = jnp.sum(zp_ref[...][:, 0, :], axis=0, keepdims=True) * (1.0 / T)
        qf = jnp.dot(z, wq_ref[...], preferred_element_type=_F32)
        m = lax.dot_general(qf, wk_ref[...], (((1,), (1,)), ((), ())),
                            preferred_element_type=_F32)
        m_ref[...] = m * 0.125

    return pl.pallas_call(
        body,
        in_specs=[
            pl.BlockSpec((NT, 1, D_A), lambda: (0, 0, 0)),
            pl.BlockSpec((D_A, N_ASPECTS * D_K), lambda: (0, 0)),
            pl.BlockSpec((D_POOL, N_ASPECTS * D_K), lambda: (0, 0)),
        ],
        out_specs=pl.BlockSpec((1, D_POOL), lambda: (0, 0)),
        out_shape=jax.ShapeDtypeStruct((1, D_POOL), _F32),
    )(zparts, wq2, wk2)


def _combined_call(pool, m):
    def body(p_ref, m_ref, o_ref):
        o_ref[...] = lax.dot_general(
            p_ref[...], m_ref[...], (((1,), (1,)), ((), ())),
            preferred_element_type=_F32)

    return pl.pallas_call(
        body,
        grid=(NPT,),
        in_specs=[
            pl.BlockSpec((PT, D_POOL), lambda i: (i, 0)),
            pl.BlockSpec((1, D_POOL), lambda i: (0, 0)),
        ],
        out_specs=pl.BlockSpec((PT, 1), lambda i: (i, 0)),
        out_shape=jax.ShapeDtypeStruct((N_POOL, 1), _F32),
    )(pool, m)


def _topk_call(c2d, lam, warm):
    rows, cols = c2d.shape

    def body(c_ref, lam_ref, warm_ref, a_ref, i_ref):
        c = c_ref[...] * lam_ref[0]
        cmax = jnp.max(c)
        e = jnp.exp(c - cmax)
        soft = e / jnp.sum(e)
        flat = (lax.broadcasted_iota(jnp.int32, (rows, cols), 0) * cols
                + lax.broadcasted_iota(jnp.int32, (rows, cols), 1))
        cur = soft
        vals = []
        for kk in range(TOP_K):
            mx = jnp.max(cur)
            am = jnp.min(jnp.where(cur == mx, flat, jnp.int32(N_POOL)))
            vals.append(mx)
            i_ref[kk] = am
            cur = jnp.where(flat == am, _F32(-1.0), cur)
        vsum = vals[0]
        for kk in range(1, TOP_K):
            vsum = vsum + vals[kk]
        warmb = warm_ref[0] != 0
        for kk in range(TOP_K):
            a_ref[kk] = jnp.where(warmb, vals[kk],
                                  vals[kk] / (vsum + 1e-9))

    return pl.pallas_call(
        body,
        in_specs=[
            pl.BlockSpec((rows, cols), lambda: (0, 0)),
            pl.BlockSpec(memory_space=pltpu.SMEM),
            pl.BlockSpec(memory_space=pltpu.SMEM),
        ],
        out_specs=[
            pl.BlockSpec(memory_space=pltpu.SMEM),
            pl.BlockSpec(memory_space=pltpu.SMEM),
        ],
        out_shape=[
            jax.ShapeDtypeStruct((TOP_K,), _F32),
            jax.ShapeDtypeStruct((TOP_K,), jnp.int32),
        ],
    )(c2d, lam, warm)


def _wm_call(au, bv, alpha16, w_base, gamma):
    def body(au_ref, bv_ref, al_ref, wb_ref, g_ref, o_ref):
        delta = jnp.dot(au_ref[...] * al_ref[...], bv_ref[...],
                        preferred_element_type=_F32)
        o_ref[...] = wb_ref[...] + g_ref[0] * delta

    return pl.pallas_call(
        body,
        in_specs=[
            pl.BlockSpec((D_B, 2 * TOP_K), lambda: (0, 0)),
            pl.BlockSpec((2 * TOP_K, D_A), lambda: (0, 0)),
            pl.BlockSpec((1, 2 * TOP_K), lambda: (0, 0)),
            pl.BlockSpec((D_B, D_A), lambda: (0, 0)),
            pl.BlockSpec(memory_space=pltpu.SMEM),
        ],
        out_specs=pl.BlockSpec((D_B, D_A), lambda: (0, 0)),
        out_shape=jax.ShapeDtypeStruct((D_B, D_A), _F32),
    )(au, bv, alpha16, w_base, gamma)


def _hmid_call(h_a, wm, b_base, s, b):
    def body(x_ref, w_ref, bb_ref, s_ref, b_ref, o_ref):
        t = lax.dot_general(x_ref[...], w_ref[...],
                            (((1,), (1,)), ((), ())),
                            preferred_element_type=_F32) + bb_ref[...]
        o_ref[...] = _ln_in(t, s_ref[...], b_ref[...])

    tile = lambda i: (i, 0)
    full = lambda i: (0, 0)
    return pl.pallas_call(
        body,
        grid=(NT,),
        in_specs=[
            pl.BlockSpec((TT, D_A), tile),
            pl.BlockSpec((D_B, D_A), full),
            pl.BlockSpec((1, D_B), full),
            pl.BlockSpec((1, D_B), full),
            pl.BlockSpec((1, D_B), full),
        ],
        out_specs=pl.BlockSpec((TT, D_B), tile),
        out_shape=jax.ShapeDtypeStruct((T, D_B), _F32),
    )(h_a, wm, b_base, s, b)


def _lmhead_call(x, w):
    def body(x_ref, w_ref, o_ref):
        o_ref[...] = jnp.dot(x_ref[...], w_ref[...],
                             preferred_element_type=_F32)

    return pl.pallas_call(
        body,
        grid=(NVT,),
        in_specs=[
            pl.BlockSpec((T, D_B), lambda j: (0, 0)),
            pl.BlockSpec((D_B, VT), lambda j: (0, j)),
        ],
        out_specs=pl.BlockSpec((T, VT), lambda j: (0, j)),
        out_shape=jax.ShapeDtypeStruct((T, VOCAB), _F32),
    )(x, w)


def kernel(input_ids, lambda_val, is_warmup, embed_table, a_ln1_s, a_ln1_b,
           a_wqkv, a_wo, a_ln2_s, a_ln2_b, a_w1, a_b1, a_w2, a_b2,
           pool_vectors, w_key, w_query, w_base, b_base, gamma, asm_ln_s,
           asm_ln_b, b_ln1_s, b_ln1_b, b_wqkv, b_wo, b_ln2_s, b_ln2_b,
           b_w1, b_b1, b_w2, b_b2, lm_head_w):
    row2 = lambda a: jnp.asarray(a, _F32).reshape(1, -1)

    ids = input_ids.reshape(T).astype(jnp.int32)
    g = _embed_gather(embed_table, ids)
    pos = _pos_enc_const(T, D_A)

    attn, x = _qkvattn_a_call(g, pos, row2(a_ln1_s), row2(a_ln1_b), a_wqkv)
    h_a, zparts = _woffn_call(x, attn, a_wo, row2(a_ln2_s), row2(a_ln2_b),
                              a_w1, row2(a_b1), a_w2, row2(a_b2),
                              want_zsum=True)

    wq2 = w_query.transpose(1, 0, 2).reshape(D_A, N_ASPECTS * D_K)
    wk2 = w_key.transpose(1, 0, 2).reshape(D_POOL, N_ASPECTS * D_K)
    m = _route_m_call(zparts, wq2, wk2)
    combined = _combined_call(pool_vectors, m)
    lam = jnp.asarray(lambda_val, _F32).reshape(1)
    warm = jnp.asarray(is_warmup, jnp.int32).reshape(1)
    alphas, indices = _topk_call(combined.reshape(64, 128), lam, warm)

    gathered = _pool_gather(pool_vectors, indices)
    au = gathered[:, :D_B * R].reshape(TOP_K, D_B, R).transpose(1, 0, 2)
    au = au.reshape(D_B, TOP_K * R)
    bv = gathered[:, D_B * R:].reshape(TOP_K * R, D_A)
    alpha16 = jnp.repeat(alphas, R).reshape(1, TOP_K * R)
    wm = _wm_call(au, bv, alpha16, w_base, gamma.reshape(1))

    h_mid = _hmid_call(h_a, wm, row2(b_base), row2(asm_ln_s), row2(asm_ln_b))

    attn2 = _qkvattn_b_call(h_mid, row2(b_ln1_s), row2(b_ln1_b), b_wqkv)
    [h_out] = _woffn_call(h_mid, attn2, b_wo, row2(b_ln2_s), row2(b_ln2_b),
                          b_w1, row2(b_b1), b_w2, row2(b_b2),
                          want_zsum=False)

    logits = _lmhead_call(h_out, lm_head_w)
    return logits.reshape(1, T, VOCAB)


# speedup vs baseline: 2.4701x; 1.0050x over previous
import functools

import jax
import jax.numpy as jnp
from jax import lax
from jax.experimental import pallas as pl
from jax.experimental.pallas import tpu as pltpu
from jax.experimental.pallas import tpu_sc as plsc

VOCAB = 32000
D_A = 768
D_B = 768
N_HEADS = 12
D_H = 64
D_FF = 3072
N_POOL = 8192
R = 2
TOP_K = 8
D_K = 64
N_ASPECTS = 4
T = 2048
D_POOL = R * (D_A + D_B)

TT = 256
NT = T // TT
PT = 1024
NPT = N_POOL // PT
VT = 1280
NVT = VOCAB // VT

_F32 = jnp.float32


def _ln_in(x, s, b):
    m = jnp.mean(x, axis=-1, keepdims=True)
    v = jnp.mean((x - m) ** 2, axis=-1, keepdims=True)
    return (x - m) * lax.rsqrt(v + 1e-5) * s + b


def _pos_enc_const(seq_len, d_model):
    pos = jnp.arange(seq_len)[:, None]
    i = jnp.arange(d_model // 2)[None, :]
    angle = pos / 10000 ** (2 * i / d_model)
    enc = jnp.concatenate([jnp.sin(angle), jnp.cos(angle)], axis=-1)
    return enc[:, :d_model].astype(_F32)


def _embed_gather(table, idx):
    info = plsc.get_sparse_core_info()
    nc, ns = info.num_cores, info.num_subcores
    nw = nc * ns
    bpw = T // nw
    mesh = plsc.VectorSubcoreMesh(core_axis_name="c", subcore_axis_name="s")

    @functools.partial(
        pl.kernel, mesh=mesh,
        out_type=jax.ShapeDtypeStruct((T, D_A), _F32),
        scratch_types=[
            pltpu.VMEM((bpw,), jnp.int32),
            pltpu.VMEM((bpw, D_A), _F32),
            pltpu.SemaphoreType.DMA,
        ],
    )
    def k(table_hbm, idx_hbm, out_hbm, idx_v, rows_v, sem):
        wid = lax.axis_index("s") * nc + lax.axis_index("c")
        base = wid * bpw
        pltpu.sync_copy(idx_hbm.at[pl.ds(base, bpw)], idx_v)
        pltpu.async_copy(table_hbm.at[idx_v], rows_v, sem).wait()
        pltpu.sync_copy(rows_v, out_hbm.at[pl.ds(base, bpw)])

    return k(table, idx)


def _pool_gather(pool, idx):
    info = plsc.get_sparse_core_info()
    nc = info.num_cores
    mesh = plsc.VectorSubcoreMesh(core_axis_name="c", subcore_axis_name="s")

    @functools.partial(
        pl.kernel, mesh=mesh,
        out_type=jax.ShapeDtypeStruct((TOP_K, D_POOL), _F32),
        scratch_types=[
            pltpu.VMEM((TOP_K,), jnp.int32),
            pltpu.VMEM((TOP_K, D_POOL), _F32),
            pltpu.SemaphoreType.DMA,
        ],
    )
    def k(pool_hbm, idx_hbm, out_hbm, idx_v, rows_v, sem):
        wid = lax.axis_index("s") * nc + lax.axis_index("c")

        @pl.when(wid == 0)
        def _():
            pltpu.sync_copy(idx_hbm, idx_v)
            pltpu.async_copy(pool_hbm.at[idx_v], rows_v, sem).wait()
            pltpu.sync_copy(rows_v, out_hbm)

    return k(pool, idx)


DP = 128
DAP = N_HEADS * DP


def _pad_heads(qkv, off):
    pieces = []
    for h in range(N_HEADS):
        pieces.append(qkv[:, off + h * D_H:off + (h + 1) * D_H])
        pieces.append(jnp.zeros((qkv.shape[0], DP - D_H), _F32))
    return jnp.concatenate(pieces, axis=1)


def _attn_inner(i, qkv, k_scr, v_scr, o_ref):
    def attn_len(L):
        row = i * TT + lax.broadcasted_iota(jnp.int32, (TT, L), 0)
        col = lax.broadcasted_iota(jnp.int32, (TT, L), 1)
        madd = jnp.where(col <= row, _F32(0.0), _F32(-1e9))
        outs = []
        for h in range(N_HEADS):
            qh = qkv[:, h * D_H:(h + 1) * D_H] * 0.125
            kh = k_scr[0:L, h * DP:(h + 1) * DP]
            vh = v_scr[0:L, h * DP:(h + 1) * DP]
            s = lax.dot_general(
                jnp.concatenate(
                    [qh, jnp.zeros((TT, DP - D_H), _F32)], axis=1),
                kh, (((1,), (1,)), ((), ())),
                preferred_element_type=_F32) + madd
            m = jnp.max(s, axis=-1, keepdims=True)
            e = jnp.exp(s - m)
            rden = 1.0 / jnp.sum(e, axis=-1, keepdims=True)
            outs.append(jnp.dot(e, vh,
                                preferred_element_type=_F32)[:, :D_H] * rden)
        o_ref[...] = jnp.concatenate(outs, axis=1)

    for pi in range(NT // 2):

        @pl.when(i // 2 == pi)
        def _(pi=pi):
            attn_len((pi + 1) * 2 * TT)


def _qkvattn_a_call(g, pos, s1, b1, wqkv):
    def body(g_ref, p_ref, s_ref, b_ref, w_ref, a_ref, x_ref, k_scr, v_scr):
        i = pl.program_id(0)
        x = g_ref[...] + p_ref[...]
        x_ref[...] = x
        h = _ln_in(x, s_ref[...], b_ref[...])
        qkv = jnp.dot(h, w_ref[...], preferred_element_type=_F32)
        k_scr[pl.ds(i * TT, TT), :] = _pad_heads(qkv, D_A)
        v_scr[pl.ds(i * TT, TT), :] = _pad_heads(qkv, 2 * D_A)

        @pl.when(i % 2 == 0)
        def _():
            k_scr[pl.ds((i + 1) * TT, TT), :] = jnp.zeros((TT, DAP), _F32)
            v_scr[pl.ds((i + 1) * TT, TT), :] = jnp.zeros((TT, DAP), _F32)

        _attn_inner(i, qkv, k_scr, v_scr, a_ref)

    tile = lambda i: (i, 0)
    full = lambda i: (0, 0)
    return pl.pallas_call(
        body,
        grid=(NT,),
        in_specs=[
            pl.BlockSpec((TT, D_A), tile),
            pl.BlockSpec((TT, D_A), tile),
            pl.BlockSpec((1, D_A), full),
            pl.BlockSpec((1, D_A), full),
            pl.BlockSpec((D_A, 3 * D_A), full),
        ],
        out_specs=[
            pl.BlockSpec((TT, D_A), tile),
            pl.BlockSpec((TT, D_A), tile),
        ],
        out_shape=[jax.ShapeDtypeStruct((T, D_A), _F32)] * 2,
        scratch_shapes=[
            pltpu.VMEM((T, DAP), _F32),
            pltpu.VMEM((T, DAP), _F32),
        ],
    )(g, pos, s1, b1, wqkv)


def _qkvattn_b_call(h_a, wm, bb, asm_s, asm_b, s1, b1, wqkv):
    def body(x_ref, wm_ref, bb_ref, as_ref, ab_ref, s_ref, b_ref, w_ref,
             a_ref, hm_ref, k_scr, v_scr):
        i = pl.program_id(0)
        t = lax.dot_general(x_ref[...], wm_ref[...],
                            (((1,), (1,)), ((), ())),
                            preferred_element_type=_F32) + bb_ref[...]
        hm = _ln_in(t, as_ref[...], ab_ref[...])
        hm_ref[...] = hm
        h = _ln_in(hm, s_ref[...], b_ref[...])
        qkv = jnp.dot(h, w_ref[...], preferred_element_type=_F32)
        k_scr[pl.ds(i * TT, TT), :] = _pad_heads(qkv, D_A)
        v_scr[pl.ds(i * TT, TT), :] = _pad_heads(qkv, 2 * D_A)

        @pl.when(i % 2 == 0)
        def _():
            k_scr[pl.ds((i + 1) * TT, TT), :] = jnp.zeros((TT, DAP), _F32)
            v_scr[pl.ds((i + 1) * TT, TT), :] = jnp.zeros((TT, DAP), _F32)

        _attn_inner(i, qkv, k_scr, v_scr, a_ref)

    tile = lambda i: (i, 0)
    full = lambda i: (0, 0)
    return pl.pallas_call(
        body,
        grid=(NT,),
        in_specs=[
            pl.BlockSpec((TT, D_A), tile),
            pl.BlockSpec((D_B, D_A), full),
            pl.BlockSpec((1, D_B), full),
            pl.BlockSpec((1, D_B), full),
            pl.BlockSpec((1, D_B), full),
            pl.BlockSpec((1, D_A), full),
            pl.BlockSpec((1, D_A), full),
            pl.BlockSpec((D_A, 3 * D_A), full),
        ],
        out_specs=[
            pl.BlockSpec((TT, D_A), tile),
            pl.BlockSpec((TT, D_B), tile),
        ],
        out_shape=[jax.ShapeDtypeStruct((T, D_A), _F32),
                   jax.ShapeDtypeStruct((T, D_B), _F32)],
        scratch_shapes=[
            pltpu.VMEM((T, DAP), _F32),
            pltpu.VMEM((T, DAP), _F32),
        ],
    )(h_a, wm, bb, asm_s, asm_b, s1, b1, wqkv)


def _woffn_a_call(x, attn, wo, s2, b2, w1, bb1, w2, bb2, wq2, wk2):
    def body(x_ref, a_ref, wo_ref, s_ref, b_ref, w1_ref, b1_ref,
             w2_ref, b2_ref, wq_ref, wk_ref, y_ref, m_ref, z_scr):
        i = pl.program_id(0)
        x1 = x_ref[...] + jnp.dot(a_ref[...], wo_ref[...],
                                  preferred_element_type=_F32)
        h2 = _ln_in(x1, s_ref[...], b_ref[...])
        ff = jax.nn.gelu(jnp.dot(h2, w1_ref[...],
                                 preferred_element_type=_F32) + b1_ref[...])
        y = x1 + jnp.dot(ff, w2_ref[...],
                         preferred_element_type=_F32) + b2_ref[...]
        y_ref[...] = y
        zp = jnp.sum(y, axis=0, keepdims=True)

        @pl.when(i == 0)
        def _():
            z_scr[...] = zp

        @pl.when(i > 0)
        def _():
            z_scr[...] = z_scr[...] + zp

        @pl.when(i == NT - 1)
        def _():
            z = z_scr[...] * (1.0 / T)
            qf = jnp.dot(z, wq_ref[...], preferred_element_type=_F32)
            m = lax.dot_general(qf, wk_ref[...], (((1,), (1,)), ((), ())),
                                preferred_element_type=_F32)
            m_ref[...] = m * 0.125

    tile = lambda i: (i, 0)
    full = lambda i: (0, 0)
    return pl.pallas_call(
        body,
        grid=(NT,),
        in_specs=[
            pl.BlockSpec((TT, D_A), tile),
            pl.BlockSpec((TT, D_A), tile),
            pl.BlockSpec((D_A, D_A), full),
            pl.BlockSpec((1, D_A), full),
            pl.BlockSpec((1, D_A), full),
            pl.BlockSpec((D_A, D_FF), full),
            pl.BlockSpec((1, D_FF), full),
            pl.BlockSpec((D_FF, D_A), full),
            pl.BlockSpec((1, D_A), full),
            pl.BlockSpec((D_A, N_ASPECTS * D_K), full),
            pl.BlockSpec((D_POOL, N_ASPECTS * D_K), full),
        ],
        out_specs=[
            pl.BlockSpec((TT, D_A), tile),
            pl.BlockSpec((1, D_POOL), full),
        ],
        out_shape=[jax.ShapeDtypeStruct((T, D_A), _F32),
                   jax.ShapeDtypeStruct((1, D_POOL), _F32)],
        scratch_shapes=[pltpu.VMEM((1, D_A), _F32)],
    )(x, attn, wo, s2, b2, w1, bb1, w2, bb2, wq2, wk2)


def _woffn_b_call(x, attn, wo, s2, b2, w1, bb1, w2, bb2):
    def body(x_ref, a_ref, wo_ref, s_ref, b_ref, w1_ref, b1_ref,
             w2_ref, b2_ref, y_ref):
        x1 = x_ref[...] + jnp.dot(a_ref[...], wo_ref[...],
                                  preferred_element_type=_F32)
        h2 = _ln_in(x1, s_ref[...], b_ref[...])
        ff = jax.nn.gelu(jnp.dot(h2, w1_ref[...],
                                 preferred_element_type=_F32) + b1_ref[...])
        y_ref[...] = x1 + jnp.dot(ff, w2_ref[...],
                                  preferred_element_type=_F32) + b2_ref[...]

    tile = lambda i: (i, 0)
    full = lambda i: (0, 0)
    return pl.pallas_call(
        body,
        grid=(NT,),
        in_specs=[
            pl.BlockSpec((TT, D_A), tile),
            pl.BlockSpec((TT, D_A), tile),
            pl.BlockSpec((D_A, D_A), full),
            pl.BlockSpec((1, D_A), full),
            pl.BlockSpec((1, D_A), full),
            pl.BlockSpec((D_A, D_FF), full),
            pl.BlockSpec((1, D_FF), full),
            pl.BlockSpec((D_FF, D_A), full),
            pl.BlockSpec((1, D_A), full),
        ],
        out_specs=pl.BlockSpec((TT, D_A), tile),
        out_shape=jax.ShapeDtypeStruct((T, D_A), _F32),
    )(x, attn, wo, s2, b2, w1, bb1, w2, bb2)


def _score_topk_call(pool, m, lam, warm):
    def body(p_ref, m_ref, lam_ref, warm_ref, a_ref, i_ref, c_scr):
        i = pl.program_id(0)
        c_scr[pl.ds(i, 1), :] = lax.dot_general(
            m_ref[...], p_ref[...], (((1,), (1,)), ((), ())),
            preferred_element_type=_F32)

        @pl.when(i == NPT - 1)
        def _():
            c = c_scr[...] * lam_ref[0]
            cmax = jnp.max(c)
            e = jnp.exp(c - cmax)
            soft = e / jnp.sum(e)
            flat = (lax.broadcasted_iota(jnp.int32, (NPT, PT), 0) * PT
                    + lax.broadcasted_iota(jnp.int32, (NPT, PT), 1))
            cur = soft
            vals = []
            for kk in range(TOP_K):
                mx = jnp.max(cur)
                am = jnp.min(jnp.where(cur == mx, flat, jnp.int32(N_POOL)))
                vals.append(mx)
                i_ref[kk] = am
                cur = jnp.where(flat == am, _F32(-1.0), cur)
            vsum = vals[0]
            for kk in range(1, TOP_K):
                vsum = vsum + vals[kk]
            warmb = warm_ref[0] != 0
            for kk in range(TOP_K):
                a_ref[kk] = jnp.where(warmb, vals[kk],
                                      vals[kk] / (vsum + 1e-9))

    return pl.pallas_call(
        body,
        grid=(NPT,),
        in_specs=[
            pl.BlockSpec((PT, D_POOL), lambda i: (i, 0)),
            pl.BlockSpec((1, D_POOL), lambda i: (0, 0)),
            pl.BlockSpec(memory_space=pltpu.SMEM),
            pl.BlockSpec(memory_space=pltpu.SMEM),
        ],
        out_specs=[
            pl.BlockSpec(memory_space=pltpu.SMEM),
            pl.BlockSpec(memory_space=pltpu.SMEM),
        ],
        out_shape=[
            jax.ShapeDtypeStruct((TOP_K,), _F32),
            jax.ShapeDtypeStruct((TOP_K,), jnp.int32),
        ],
        scratch_shapes=[pltpu.VMEM((NPT, PT), _F32)],
    )(pool, m, lam, warm)


def _wm_call(au, bv, alpha16, w_base, gamma):
    def body(au_ref, bv_ref, al_ref, wb_ref, g_ref, o_ref):
        delta = jnp.dot(au_ref[...] * al_ref[...], bv_ref[...],
                        preferred_element_type=_F32)
        o_ref[...] = wb_ref[...] + g_ref[0] * delta

    return pl.pallas_call(
        body,
        in_specs=[
            pl.BlockSpec((D_B, 2 * TOP_K), lambda: (0, 0)),
            pl.BlockSpec((2 * TOP_K, D_A), lambda: (0, 0)),
            pl.BlockSpec((1, 2 * TOP_K), lambda: (0, 0)),
            pl.BlockSpec((D_B, D_A), lambda: (0, 0)),
            pl.BlockSpec(memory_space=pltpu.SMEM),
        ],
        out_specs=pl.BlockSpec((D_B, D_A), lambda: (0, 0)),
        out_shape=jax.ShapeDtypeStruct((D_B, D_A), _F32),
    )(au, bv, alpha16, w_base, gamma)


def _lmhead_call(x, w):
    def body(x_ref, w_ref, o_ref):
        o_ref[...] = jnp.dot(x_ref[...], w_ref[...],
                             preferred_element_type=_F32)

    return pl.pallas_call(
        body,
        grid=(NVT,),
        in_specs=[
            pl.BlockSpec((T, D_B), lambda j: (0, 0)),
            pl.BlockSpec((D_B, VT), lambda j: (0, j)),
        ],
        out_specs=pl.BlockSpec((T, VT), lambda j: (0, j)),
        out_shape=jax.ShapeDtypeStruct((T, VOCAB), _F32),
    )(x, w)


def kernel(input_ids, lambda_val, is_warmup, embed_table, a_ln1_s, a_ln1_b,
           a_wqkv, a_wo, a_ln2_s, a_ln2_b, a_w1, a_b1, a_w2, a_b2,
           pool_vectors, w_key, w_query, w_base, b_base, gamma, asm_ln_s,
           asm_ln_b, b_ln1_s, b_ln1_b, b_wqkv, b_wo, b_ln2_s, b_ln2_b,
           b_w1, b_b1, b_w2, b_b2, lm_head_w):
    row2 = lambda a: jnp.asarray(a, _F32).reshape(1, -1)

    ids = input_ids.reshape(T).astype(jnp.int32)
    g = _embed_gather(embed_table, ids)
    pos = _pos_enc_const(T, D_A)

    attn, x = _qkvattn_a_call(g, pos, row2(a_ln1_s), row2(a_ln1_b), a_wqkv)
    wq2 = w_query.transpose(1, 0, 2).reshape(D_A, N_ASPECTS * D_K)
    wk2 = w_key.transpose(1, 0, 2).reshape(D_POOL, N_ASPECTS * D_K)
    h_a, m = _woffn_a_call(x, attn, a_wo, row2(a_ln2_s), row2(a_ln2_b),
                           a_w1, row2(a_b1), a_w2, row2(a_b2), wq2, wk2)

    lam = jnp.asarray(lambda_val, _F32).reshape(1)
    warm = jnp.asarray(is_warmup, jnp.int32).reshape(1)
    alphas, indices = _score_topk_call(pool_vectors, m, lam, warm)

    gathered = _pool_gather(pool_vectors, indices)
    au = gathered[:, :D_B * R].reshape(TOP_K, D_B, R).transpose(1, 0, 2)
    au = au.reshape(D_B, TOP_K * R)
    bv = gathered[:, D_B * R:].reshape(TOP_K * R, D_A)
    alpha16 = jnp.repeat(alphas, R).reshape(1, TOP_K * R)
    wm = _wm_call(au, bv, alpha16, w_base, gamma.reshape(1))

    attn2, h_mid = _qkvattn_b_call(h_a, wm, row2(b_base), row2(asm_ln_s),
                                   row2(asm_ln_b), row2(b_ln1_s),
                                   row2(b_ln1_b), b_wqkv)
    h_out = _woffn_b_call(h_mid, attn2, b_wo, row2(b_ln2_s), row2(b_ln2_b),
                          b_w1, row2(b_b1), b_w2, row2(b_b2))

    logits = _lmhead_call(h_out, lm_head_w)
    return logits.reshape(1, T, VOCAB)


# speedup vs baseline: 2.6364x; 1.0673x over previous
import functools

import jax
import jax.numpy as jnp
import numpy as np
from jax import lax
from jax.experimental import pallas as pl
from jax.experimental.pallas import tpu as pltpu
from jax.experimental.pallas import tpu_sc as plsc

VOCAB = 32000
D_A = 768
D_B = 768
N_HEADS = 12
D_H = 64
D_FF = 3072
N_POOL = 8192
R = 2
TOP_K = 8
D_K = 64
N_ASPECTS = 4
T = 2048
D_POOL = R * (D_A + D_B)

TT = 256
NT = T // TT
PT = 1024
NPT = N_POOL // PT
VT = 1280
NVT = VOCAB // VT

_F32 = jnp.float32


def _ln_in(x, s, b):
    m = jnp.mean(x, axis=-1, keepdims=True)
    v = jnp.mean((x - m) ** 2, axis=-1, keepdims=True)
    return (x - m) * lax.rsqrt(v + 1e-5) * s + b


def _pos_enc_const(seq_len, d_model):
    pos = np.arange(seq_len, dtype=np.float32)[:, None]
    i = np.arange(d_model // 2, dtype=np.float32)[None, :]
    angle = (pos / (10000.0 ** (2.0 * i / d_model))).astype(np.float32)
    enc = np.concatenate([np.sin(angle), np.cos(angle)], axis=-1)
    return enc[:, :d_model].astype(np.float32)


_POS_ENC = _pos_enc_const(T, D_A)


def _embed_gather(table, idx):
    info = plsc.get_sparse_core_info()
    nc, ns = info.num_cores, info.num_subcores
    nw = nc * ns
    bpw = T // nw
    mesh = plsc.VectorSubcoreMesh(core_axis_name="c", subcore_axis_name="s")

    @functools.partial(
        pl.kernel, mesh=mesh,
        out_type=jax.ShapeDtypeStruct((T, D_A), _F32),
        scratch_types=[
            pltpu.VMEM((bpw,), jnp.int32),
            pltpu.VMEM((bpw, D_A), _F32),
            pltpu.SemaphoreType.DMA,
        ],
    )
    def k(table_hbm, idx_hbm, out_hbm, idx_v, rows_v, sem):
        wid = lax.axis_index("s") * nc + lax.axis_index("c")
        base = wid * bpw
        pltpu.sync_copy(idx_hbm.at[pl.ds(base, bpw)], idx_v)
        pltpu.async_copy(table_hbm.at[idx_v], rows_v, sem).wait()
        pltpu.sync_copy(rows_v, out_hbm.at[pl.ds(base, bpw)])

    return k(table, idx)


def _pool_gather(pool, idx):
    info = plsc.get_sparse_core_info()
    nc = info.num_cores
    mesh = plsc.VectorSubcoreMesh(core_axis_name="c", subcore_axis_name="s")

    @functools.partial(
        pl.kernel, mesh=mesh,
        out_type=jax.ShapeDtypeStruct((TOP_K, D_POOL), _F32),
        scratch_types=[
            pltpu.VMEM((TOP_K,), jnp.int32),
            pltpu.VMEM((TOP_K, D_POOL), _F32),
            pltpu.SemaphoreType.DMA,
        ],
    )
    def k(pool_hbm, idx_hbm, out_hbm, idx_v, rows_v, sem):
        wid = lax.axis_index("s") * nc + lax.axis_index("c")

        @pl.when(wid == 0)
        def _():
            pltpu.sync_copy(idx_hbm, idx_v)
            pltpu.async_copy(pool_hbm.at[idx_v], rows_v, sem).wait()
            pltpu.sync_copy(rows_v, out_hbm)

    return k(pool, idx)


DP = 128
DAP = N_HEADS * DP


def _pad_heads(qkv, off):
    pieces = []
    for h in range(N_HEADS):
        pieces.append(qkv[:, off + h * D_H:off + (h + 1) * D_H])
        pieces.append(jnp.zeros((qkv.shape[0], DP - D_H), _F32))
    return jnp.concatenate(pieces, axis=1)


def _attn_inner(i, qkv, k_scr, v_scr, o_ref):
    def attn_len(L):
        row = i * TT + lax.broadcasted_iota(jnp.int32, (TT, L), 0)
        col = lax.broadcasted_iota(jnp.int32, (TT, L), 1)
        madd = jnp.where(col <= row, _F32(0.0), _F32(-1e9))
        outs = []
        for h in range(N_HEADS):
            qh = qkv[:, h * D_H:(h + 1) * D_H] * 0.125
            kh = k_scr[0:L, h * DP:(h + 1) * DP]
            vh = v_scr[0:L, h * DP:(h + 1) * DP]
            s = lax.dot_general(
                jnp.concatenate(
                    [qh, jnp.zeros((TT, DP - D_H), _F32)], axis=1),
                kh, (((1,), (1,)), ((), ())),
                preferred_element_type=_F32) + madd
            m = jnp.max(s, axis=-1, keepdims=True)
            e = jnp.exp(s - m)
            rden = 1.0 / jnp.sum(e, axis=-1, keepdims=True)
            outs.append(jnp.dot(e, vh,
                                preferred_element_type=_F32)[:, :D_H] * rden)
        o_ref[...] = jnp.concatenate(outs, axis=1)

    for pi in range(NT // 2):

        @pl.when(i // 2 == pi)
        def _(pi=pi):
            attn_len((pi + 1) * 2 * TT)


def _qkvattn_a_call(g, pos, s1, b1, wqkv):
    def body(g_ref, p_ref, s_ref, b_ref, w_ref, a_ref, x_ref, k_scr, v_scr):
        i = pl.program_id(0)
        x = g_ref[...] + p_ref[...]
        x_ref[...] = x
        h = _ln_in(x, s_ref[...], b_ref[...])
        qkv = jnp.dot(h, w_ref[...], preferred_element_type=_F32)
        k_scr[pl.ds(i * TT, TT), :] = _pad_heads(qkv, D_A)
        v_scr[pl.ds(i * TT, TT), :] = _pad_heads(qkv, 2 * D_A)

        @pl.when(i % 2 == 0)
        def _():
            k_scr[pl.ds((i + 1) * TT, TT), :] = jnp.zeros((TT, DAP), _F32)
            v_scr[pl.ds((i + 1) * TT, TT), :] = jnp.zeros((TT, DAP), _F32)

        _attn_inner(i, qkv, k_scr, v_scr, a_ref)

    tile = lambda i: (i, 0)
    full = lambda i: (0, 0)
    return pl.pallas_call(
        body,
        grid=(NT,),
        in_specs=[
            pl.BlockSpec((TT, D_A), tile),
            pl.BlockSpec((TT, D_A), tile),
            pl.BlockSpec((1, D_A), full),
            pl.BlockSpec((1, D_A), full),
            pl.BlockSpec((D_A, 3 * D_A), full),
        ],
        out_specs=[
            pl.BlockSpec((TT, D_A), tile),
            pl.BlockSpec((TT, D_A), tile),
        ],
        out_shape=[jax.ShapeDtypeStruct((T, D_A), _F32)] * 2,
        scratch_shapes=[
            pltpu.VMEM((T, DAP), _F32),
            pltpu.VMEM((T, DAP), _F32),
        ],
    )(g, pos, s1, b1, wqkv)


def _qkvattn_b_call(h_a, wm, bb, asm_s, asm_b, s1, b1, wqkv):
    def body(x_ref, wm_ref, bb_ref, as_ref, ab_ref, s_ref, b_ref, w_ref,
             a_ref, hm_ref, k_scr, v_scr):
        i = pl.program_id(0)
        t = lax.dot_general(x_ref[...], wm_ref[...],
                            (((1,), (1,)), ((), ())),
                            preferred_element_type=_F32) + bb_ref[...]
        hm = _ln_in(t, as_ref[...], ab_ref[...])
        hm_ref[...] = hm
        h = _ln_in(hm, s_ref[...], b_ref[...])
        qkv = jnp.dot(h, w_ref[...], preferred_element_type=_F32)
        k_scr[pl.ds(i * TT, TT), :] = _pad_heads(qkv, D_A)
        v_scr[pl.ds(i * TT, TT), :] = _pad_heads(qkv, 2 * D_A)

        @pl.when(i % 2 == 0)
        def _():
            k_scr[pl.ds((i + 1) * TT, TT), :] = jnp.zeros((TT, DAP), _F32)
            v_scr[pl.ds((i + 1) * TT, TT), :] = jnp.zeros((TT, DAP), _F32)

        _attn_inner(i, qkv, k_scr, v_scr, a_ref)

    tile = lambda i: (i, 0)
    full = lambda i: (0, 0)
    return pl.pallas_call(
        body,
        grid=(NT,),
        in_specs=[
            pl.BlockSpec((TT, D_A), tile),
            pl.BlockSpec((D_B, D_A), full),
            pl.BlockSpec((1, D_B), full),
            pl.BlockSpec((1, D_B), full),
            pl.BlockSpec((1, D_B), full),
            pl.BlockSpec((1, D_A), full),
            pl.BlockSpec((1, D_A), full),
            pl.BlockSpec((D_A, 3 * D_A), full),
        ],
        out_specs=[
            pl.BlockSpec((TT, D_A), tile),
            pl.BlockSpec((TT, D_B), tile),
        ],
        out_shape=[jax.ShapeDtypeStruct((T, D_A), _F32),
                   jax.ShapeDtypeStruct((T, D_B), _F32)],
        scratch_shapes=[
            pltpu.VMEM((T, DAP), _F32),
            pltpu.VMEM((T, DAP), _F32),
        ],
    )(h_a, wm, bb, asm_s, asm_b, s1, b1, wqkv)


def _woffn_a_call(x, attn, wo, s2, b2, w1, bb1, w2, bb2, wq2, wk2):
    def body(x_ref, a_ref, wo_ref, s_ref, b_ref, w1_ref, b1_ref,
             w2_ref, b2_ref, wq_ref, wk_ref, y_ref, m_ref, z_scr):
        i = pl.program_id(0)
        x1 = x_ref[...] + jnp.dot(a_ref[...], wo_ref[...],
                                  preferred_element_type=_F32)
        h2 = _ln_in(x1, s_ref[...], b_ref[...])
        ff = jax.nn.gelu(jnp.dot(h2, w1_ref[...],
                                 preferred_element_type=_F32) + b1_ref[...])
        y = x1 + jnp.dot(ff, w2_ref[...],
                         preferred_element_type=_F32) + b2_ref[...]
        y_ref[...] = y
        zp = jnp.sum(y, axis=0, keepdims=True)

        @pl.when(i == 0)
        def _():
            z_scr[...] = zp

        @pl.when(i > 0)
        def _():
            z_scr[...] = z_scr[...] + zp

        @pl.when(i == NT - 1)
        def _():
            z = z_scr[...] * (1.0 / T)
            qf = jnp.dot(z, wq_ref[...], preferred_element_type=_F32)
            m = lax.dot_general(qf, wk_ref[...], (((1,), (1,)), ((), ())),
                                preferred_element_type=_F32)
            m_ref[...] = m * 0.125

    tile = lambda i: (i, 0)
    full = lambda i: (0, 0)
    return pl.pallas_call(
        body,
        grid=(NT,),
        in_specs=[
            pl.BlockSpec((TT, D_A), tile),
            pl.BlockSpec((TT, D_A), tile),
            pl.BlockSpec((D_A, D_A), full),
            pl.BlockSpec((1, D_A), full),
            pl.BlockSpec((1, D_A), full),
            pl.BlockSpec((D_A, D_FF), full),
            pl.BlockSpec((1, D_FF), full),
            pl.BlockSpec((D_FF, D_A), full),
            pl.BlockSpec((1, D_A), full),
            pl.BlockSpec((D_A, N_ASPECTS * D_K), full),
            pl.BlockSpec((D_POOL, N_ASPECTS * D_K), full),
        ],
        out_specs=[
            pl.BlockSpec((TT, D_A), tile),
            pl.BlockSpec((1, D_POOL), full),
        ],
        out_shape=[jax.ShapeDtypeStruct((T, D_A), _F32),
                   jax.ShapeDtypeStruct((1, D_POOL), _F32)],
        scratch_shapes=[pltpu.VMEM((1, D_A), _F32)],
    )(x, attn, wo, s2, b2, w1, bb1, w2, bb2, wq2, wk2)


def _woffn_b_call(x, attn, wo, s2, b2, w1, bb1, w2, bb2):
    def body(x_ref, a_ref, wo_ref, s_ref, b_ref, w1_ref, b1_ref,
             w2_ref, b2_ref, y_ref):
        x1 = x_ref[...] + jnp.dot(a_ref[...], wo_ref[...],
                                  preferred_element_type=_F32)
        h2 = _ln_in(x1, s_ref[...], b_ref[...])
        ff = jax.nn.gelu(jnp.dot(h2, w1_ref[...],
                                 preferred_element_type=_F32) + b1_ref[...])
        y_ref[...] = x1 + jnp.dot(ff, w2_ref[...],
                                  preferred_element_type=_F32) + b2_ref[...]

    tile = lambda i: (i, 0)
    full = lambda i: (0, 0)
    return pl.pallas_call(
        body,
        grid=(NT,),
        in_specs=[
            pl.BlockSpec((TT, D_A), tile),
            pl.BlockSpec((TT, D_A), tile),
            pl.BlockSpec((D_A, D_A), full),
            pl.BlockSpec((1, D_A), full),
            pl.BlockSpec((1, D_A), full),
            pl.BlockSpec((D_A, D_FF), full),
            pl.BlockSpec((1, D_FF), full),
            pl.BlockSpec((D_FF, D_A), full),
            pl.BlockSpec((1, D_A), full),
        ],
        out_specs=pl.BlockSpec((TT, D_A), tile),
        out_shape=jax.ShapeDtypeStruct((T, D_A), _F32),
    )(x, attn, wo, s2, b2, w1, bb1, w2, bb2)


def _score_topk_call(pool, m, lam, warm):
    def body(p_ref, m_ref, lam_ref, warm_ref, a_ref, i_ref, c_scr):
        i = pl.program_id(0)
        c_scr[pl.ds(i, 1), :] = lax.dot_general(
            m_ref[...], p_ref[...], (((1,), (1,)), ((), ())),
            preferred_element_type=_F32)

        @pl.when(i == NPT - 1)
        def _():
            c = c_scr[...] * lam_ref[0]
            cmax = jnp.max(c)
            e = jnp.exp(c - cmax)
            soft = e / jnp.sum(e)
            flat = (lax.broadcasted_iota(jnp.int32, (NPT, PT), 0) * PT
                    + lax.broadcasted_iota(jnp.int32, (NPT, PT), 1))
            cur = soft
            vals = []
            for kk in range(TOP_K):
                mx = jnp.max(cur)
                am = jnp.min(jnp.where(cur == mx, flat, jnp.int32(N_POOL)))
                vals.append(mx)
                i_ref[kk] = am
                cur = jnp.where(flat == am, _F32(-1.0), cur)
            vsum = vals[0]
            for kk in range(1, TOP_K):
                vsum = vsum + vals[kk]
            warmb = warm_ref[0] != 0
            for kk in range(TOP_K):
                a_ref[kk] = jnp.where(warmb, vals[kk],
                                      vals[kk] / (vsum + 1e-9))

    return pl.pallas_call(
        body,
        grid=(NPT,),
        in_specs=[
            pl.BlockSpec((PT, D_POOL), lambda i: (i, 0)),
            pl.BlockSpec((1, D_POOL), lambda i: (0, 0)),
            pl.BlockSpec(memory_space=pltpu.SMEM),
            pl.BlockSpec(memory_space=pltpu.SMEM),
        ],
        out_specs=[
            pl.BlockSpec(memory_space=pltpu.SMEM),
            pl.BlockSpec(memory_space=pltpu.SMEM),
        ],
        out_shape=[
            jax.ShapeDtypeStruct((TOP_K,), _F32),
            jax.ShapeDtypeStruct((TOP_K,), jnp.int32),
        ],
        scratch_shapes=[pltpu.VMEM((NPT, PT), _F32)],
    )(pool, m, lam, warm)


def _wm_call(au, bv, alpha16, w_base, gamma):
    def body(au_ref, bv_ref, al_ref, wb_ref, g_ref, o_ref):
        delta = jnp.dot(au_ref[...] * al_ref[...], bv_ref[...],
                        preferred_element_type=_F32)
        o_ref[...] = wb_ref[...] + g_ref[0] * delta

    return pl.pallas_call(
        body,
        in_specs=[
            pl.BlockSpec((D_B, 2 * TOP_K), lambda: (0, 0)),
            pl.BlockSpec((2 * TOP_K, D_A), lambda: (0, 0)),
            pl.BlockSpec((1, 2 * TOP_K), lambda: (0, 0)),
            pl.BlockSpec((D_B, D_A), lambda: (0, 0)),
            pl.BlockSpec(memory_space=pltpu.SMEM),
        ],
        out_specs=pl.BlockSpec((D_B, D_A), lambda: (0, 0)),
        out_shape=jax.ShapeDtypeStruct((D_B, D_A), _F32),
    )(au, bv, alpha16, w_base, gamma)


def _lmhead_call(x, w):
    def body(x_ref, w_ref, o_ref):
        o_ref[...] = jnp.dot(x_ref[...], w_ref[...],
                             preferred_element_type=_F32)

    return pl.pallas_call(
        body,
        grid=(NVT,),
        in_specs=[
            pl.BlockSpec((T, D_B), lambda j: (0, 0)),
            pl.BlockSpec((D_B, VT), lambda j: (0, j)),
        ],
        out_specs=pl.BlockSpec((T, VT), lambda j: (0, j)),
        out_shape=jax.ShapeDtypeStruct((T, VOCAB), _F32),
    )(x, w)


def kernel(input_ids, lambda_val, is_warmup, embed_table, a_ln1_s, a_ln1_b,
           a_wqkv, a_wo, a_ln2_s, a_ln2_b, a_w1, a_b1, a_w2, a_b2,
           pool_vectors, w_key, w_query, w_base, b_base, gamma, asm_ln_s,
           asm_ln_b, b_ln1_s, b_ln1_b, b_wqkv, b_wo, b_ln2_s, b_ln2_b,
           b_w1, b_b1, b_w2, b_b2, lm_head_w):
    row2 = lambda a: jnp.asarray(a, _F32).reshape(1, -1)

    ids = input_ids.reshape(T).astype(jnp.int32)
    g = _embed_gather(embed_table, ids)
    pos = jnp.asarray(_POS_ENC)

    attn, x = _qkvattn_a_call(g, pos, row2(a_ln1_s), row2(a_ln1_b), a_wqkv)
    wq2 = w_query.transpose(1, 0, 2).reshape(D_A, N_ASPECTS * D_K)
    wk2 = w_key.transpose(1, 0, 2).reshape(D_POOL, N_ASPECTS * D_K)
    h_a, m = _woffn_a_call(x, attn, a_wo, row2(a_ln2_s), row2(a_ln2_b),
                           a_w1, row2(a_b1), a_w2, row2(a_b2), wq2, wk2)

    lam = jnp.asarray(lambda_val, _F32).reshape(1)
    warm = jnp.asarray(is_warmup, jnp.int32).reshape(1)
    alphas, indices = _score_topk_call(pool_vectors, m, lam, warm)

    gathered = _pool_gather(pool_vectors, indices)
    au = gathered[:, :D_B * R].reshape(TOP_K, D_B, R).transpose(1, 0, 2)
    au = au.reshape(D_B, TOP_K * R)
    bv = gathered[:, D_B * R:].reshape(TOP_K * R, D_A)
    alpha16 = jnp.repeat(alphas, R).reshape(1, TOP_K * R)
    wm = _wm_call(au, bv, alpha16, w_base, gamma.reshape(1))

    attn2, h_mid = _qkvattn_b_call(h_a, wm, row2(b_base), row2(asm_ln_s),
                                   row2(asm_ln_b), row2(b_ln1_s),
                                   row2(b_ln1_b), b_wqkv)
    h_out = _woffn_b_call(h_mid, attn2, b_wo, row2(b_ln2_s), row2(b_ln2_b),
                          b_w1, row2(b_b1), b_w2, row2(b_b2))

    logits = _lmhead_call(h_out, lm_head_w)
    return logits.reshape(1, T, VOCAB)


# speedup vs baseline: 2.6549x; 1.0070x over previous
import functools

import jax
import jax.numpy as jnp
import numpy as np
from jax import lax
from jax.experimental import pallas as pl
from jax.experimental.pallas import tpu as pltpu
from jax.experimental.pallas import tpu_sc as plsc

VOCAB = 32000
D_A = 768
D_B = 768
N_HEADS = 12
D_H = 64
D_FF = 3072
N_POOL = 8192
R = 2
TOP_K = 8
D_K = 64
N_ASPECTS = 4
T = 2048
D_POOL = R * (D_A + D_B)

TT = 256
NT = T // TT
PT = 1024
NPT = N_POOL // PT
VT = 1280
NVT = VOCAB // VT

_F32 = jnp.float32


def _ln_in(x, s, b):
    m = jnp.mean(x, axis=-1, keepdims=True)
    v = jnp.mean((x - m) ** 2, axis=-1, keepdims=True)
    return (x - m) * lax.rsqrt(v + 1e-5) * s + b


def _pos_enc_const(seq_len, d_model):
    pos = np.arange(seq_len, dtype=np.float32)[:, None]
    i = np.arange(d_model // 2, dtype=np.float32)[None, :]
    angle = (pos / (10000.0 ** (2.0 * i / d_model))).astype(np.float32)
    enc = np.concatenate([np.sin(angle), np.cos(angle)], axis=-1)
    return enc[:, :d_model].astype(np.float32)


_POS_ENC = _pos_enc_const(T, D_A)


def _embed_gather(table, idx):
    info = plsc.get_sparse_core_info()
    nc, ns = info.num_cores, info.num_subcores
    nw = nc * ns
    bpw = T // nw
    mesh = plsc.VectorSubcoreMesh(core_axis_name="c", subcore_axis_name="s")

    @functools.partial(
        pl.kernel, mesh=mesh,
        out_type=jax.ShapeDtypeStruct((T, D_A), _F32),
        scratch_types=[
            pltpu.VMEM((bpw,), jnp.int32),
            pltpu.VMEM((bpw, D_A), _F32),
            pltpu.SemaphoreType.DMA,
        ],
    )
    def k(table_hbm, idx_hbm, out_hbm, idx_v, rows_v, sem):
        wid = lax.axis_index("s") * nc + lax.axis_index("c")
        base = wid * bpw
        pltpu.sync_copy(idx_hbm.at[pl.ds(base, bpw)], idx_v)
        pltpu.async_copy(table_hbm.at[idx_v], rows_v, sem).wait()
        pltpu.sync_copy(rows_v, out_hbm.at[pl.ds(base, bpw)])

    return k(table, idx)


def _pool_gather(pool, idx):
    info = plsc.get_sparse_core_info()
    nc = info.num_cores
    mesh = plsc.VectorSubcoreMesh(core_axis_name="c", subcore_axis_name="s")

    @functools.partial(
        pl.kernel, mesh=mesh,
        out_type=jax.ShapeDtypeStruct((TOP_K, D_POOL), _F32),
        scratch_types=[
            pltpu.VMEM((TOP_K,), jnp.int32),
            pltpu.VMEM((TOP_K, D_POOL), _F32),
            pltpu.SemaphoreType.DMA,
        ],
    )
    def k(pool_hbm, idx_hbm, out_hbm, idx_v, rows_v, sem):
        wid = lax.axis_index("s") * nc + lax.axis_index("c")

        @pl.when(wid == 0)
        def _():
            pltpu.sync_copy(idx_hbm, idx_v)
            pltpu.async_copy(pool_hbm.at[idx_v], rows_v, sem).wait()
            pltpu.sync_copy(rows_v, out_hbm)

    return k(pool, idx)


DP = 128
DAP = N_HEADS * DP


def _pad_heads(qkv, off):
    pieces = []
    for h in range(N_HEADS):
        pieces.append(qkv[:, off + h * D_H:off + (h + 1) * D_H])
        pieces.append(jnp.zeros((qkv.shape[0], DP - D_H), _F32))
    return jnp.concatenate(pieces, axis=1)


def _attn_inner(i, qkv, k_scr, v_scr, o_ref):
    def attn_len(L):
        row = i * TT + lax.broadcasted_iota(jnp.int32, (TT, L), 0)
        col = lax.broadcasted_iota(jnp.int32, (TT, L), 1)
        madd = jnp.where(col <= row, _F32(0.0), _F32(-1e9))
        outs = []
        for h in range(N_HEADS):
            qh = qkv[:, h * D_H:(h + 1) * D_H] * 0.125
            kh = k_scr[0:L, h * DP:(h + 1) * DP]
            vh = v_scr[0:L, h * DP:(h + 1) * DP]
            s = lax.dot_general(
                jnp.concatenate(
                    [qh, jnp.zeros((TT, DP - D_H), _F32)], axis=1),
                kh, (((1,), (1,)), ((), ())),
                preferred_element_type=_F32) + madd
            m = jnp.max(s, axis=-1, keepdims=True)
            e = jnp.exp(s - m)
            rden = 1.0 / jnp.sum(e, axis=-1, keepdims=True)
            outs.append(jnp.dot(e, vh,
                                preferred_element_type=_F32)[:, :D_H] * rden)
        o_ref[...] = jnp.concatenate(outs, axis=1)

    for pi in range(NT // 2):

        @pl.when(i // 2 == pi)
        def _(pi=pi):
            attn_len((pi + 1) * 2 * TT)


def _qkvattn_a_call(g, pos, s1, b1, wqkv):
    def body(g_ref, p_ref, s_ref, b_ref, w_ref, a_ref, x_ref, k_scr, v_scr):
        i = pl.program_id(0)
        x = g_ref[...] + p_ref[...]
        x_ref[...] = x
        h = _ln_in(x, s_ref[...], b_ref[...])
        qkv = jnp.dot(h, w_ref[...], preferred_element_type=_F32)
        k_scr[pl.ds(i * TT, TT), :] = _pad_heads(qkv, D_A)
        v_scr[pl.ds(i * TT, TT), :] = _pad_heads(qkv, 2 * D_A)

        @pl.when(i % 2 == 0)
        def _():
            k_scr[pl.ds((i + 1) * TT, TT), :] = jnp.zeros((TT, DAP), _F32)
            v_scr[pl.ds((i + 1) * TT, TT), :] = jnp.zeros((TT, DAP), _F32)

        _attn_inner(i, qkv, k_scr, v_scr, a_ref)

    tile = lambda i: (i, 0)
    full = lambda i: (0, 0)
    return pl.pallas_call(
        body,
        grid=(NT,),
        in_specs=[
            pl.BlockSpec((TT, D_A), tile),
            pl.BlockSpec((TT, D_A), tile),
            pl.BlockSpec((1, D_A), full),
            pl.BlockSpec((1, D_A), full),
            pl.BlockSpec((D_A, 3 * D_A), full),
        ],
        out_specs=[
            pl.BlockSpec((TT, D_A), tile),
            pl.BlockSpec((TT, D_A), tile),
        ],
        out_shape=[jax.ShapeDtypeStruct((T, D_A), _F32)] * 2,
        scratch_shapes=[
            pltpu.VMEM((T, DAP), _F32),
            pltpu.VMEM((T, DAP), _F32),
        ],
    )(g, pos, s1, b1, wqkv)


def _qkvattn_b_call(h_a, au, bv, alpha16, w_base, gamma, bb, asm_s, asm_b,
                    s1, b1, wqkv):
    def body(x_ref, au_ref, bv_ref, al_ref, wb_ref, g_ref, bb_ref, as_ref,
             ab_ref, s_ref, b_ref, w_ref, a_ref, hm_ref, k_scr, v_scr,
             wm_scr):
        i = pl.program_id(0)

        @pl.when(i == 0)
        def _():
            delta = jnp.dot(au_ref[...] * al_ref[...], bv_ref[...],
                            preferred_element_type=_F32)
            wm_scr[...] = wb_ref[...] + g_ref[0] * delta

        t = lax.dot_general(x_ref[...], wm_scr[...],
                            (((1,), (1,)), ((), ())),
                            preferred_element_type=_F32) + bb_ref[...]
        hm = _ln_in(t, as_ref[...], ab_ref[...])
        hm_ref[...] = hm
        h = _ln_in(hm, s_ref[...], b_ref[...])
        qkv = jnp.dot(h, w_ref[...], preferred_element_type=_F32)
        k_scr[pl.ds(i * TT, TT), :] = _pad_heads(qkv, D_A)
        v_scr[pl.ds(i * TT, TT), :] = _pad_heads(qkv, 2 * D_A)

        @pl.when(i % 2 == 0)
        def _():
            k_scr[pl.ds((i + 1) * TT, TT), :] = jnp.zeros((TT, DAP), _F32)
            v_scr[pl.ds((i + 1) * TT, TT), :] = jnp.zeros((TT, DAP), _F32)

        _attn_inner(i, qkv, k_scr, v_scr, a_ref)

    tile = lambda i: (i, 0)
    full = lambda i: (0, 0)
    return pl.pallas_call(
        body,
        grid=(NT,),
        in_specs=[
            pl.BlockSpec((TT, D_A), tile),
            pl.BlockSpec((D_B, 2 * TOP_K), full),
            pl.BlockSpec((2 * TOP_K, D_A), full),
            pl.BlockSpec((1, 2 * TOP_K), full),
            pl.BlockSpec((D_B, D_A), full),
            pl.BlockSpec(memory_space=pltpu.SMEM),
            pl.BlockSpec((1, D_B), full),
            pl.BlockSpec((1, D_B), full),
            pl.BlockSpec((1, D_B), full),
            pl.BlockSpec((1, D_A), full),
            pl.BlockSpec((1, D_A), full),
            pl.BlockSpec((D_A, 3 * D_A), full),
        ],
        out_specs=[
            pl.BlockSpec((TT, D_A), tile),
            pl.BlockSpec((TT, D_B), tile),
        ],
        out_shape=[jax.ShapeDtypeStruct((T, D_A), _F32),
                   jax.ShapeDtypeStruct((T, D_B), _F32)],
        scratch_shapes=[
            pltpu.VMEM((T, DAP), _F32),
            pltpu.VMEM((T, DAP), _F32),
            pltpu.VMEM((D_B, D_A), _F32),
        ],
    )(h_a, au, bv, alpha16, w_base, gamma, bb, asm_s, asm_b, s1, b1, wqkv)


def _woffn_a_call(x, attn, wo, s2, b2, w1, bb1, w2, bb2, wq2, wk2):
    def body(x_ref, a_ref, wo_ref, s_ref, b_ref, w1_ref, b1_ref,
             w2_ref, b2_ref, wq_ref, wk_ref, y_ref, m_ref, z_scr):
        i = pl.program_id(0)
        x1 = x_ref[...] + jnp.dot(a_ref[...], wo_ref[...],
                                  preferred_element_type=_F32)
        h2 = _ln_in(x1, s_ref[...], b_ref[...])
        ff = jax.nn.gelu(jnp.dot(h2, w1_ref[...],
                                 preferred_element_type=_F32) + b1_ref[...])
        y = x1 + jnp.dot(ff, w2_ref[...],
                         preferred_element_type=_F32) + b2_ref[...]
        y_ref[...] = y
        zp = jnp.sum(y, axis=0, keepdims=True)

        @pl.when(i == 0)
        def _():
            z_scr[...] = zp

        @pl.when(i > 0)
        def _():
            z_scr[...] = z_scr[...] + zp

        @pl.when(i == NT - 1)
        def _():
            z = z_scr[...] * (1.0 / T)
            qf = jnp.dot(z, wq_ref[...], preferred_element_type=_F32)
            m = lax.dot_general(qf, wk_ref[...], (((1,), (1,)), ((), ())),
                                preferred_element_type=_F32)
            m_ref[...] = m * 0.125

    tile = lambda i: (i, 0)
    full = lambda i: (0, 0)
    return pl.pallas_call(
        body,
        grid=(NT,),
        in_specs=[
            pl.BlockSpec((TT, D_A), tile),
            pl.BlockSpec((TT, D_A), tile),
            pl.BlockSpec((D_A, D_A), full),
            pl.BlockSpec((1, D_A), full),
            pl.BlockSpec((1, D_A), full),
            pl.BlockSpec((D_A, D_FF), full),
            pl.BlockSpec((1, D_FF), full),
            pl.BlockSpec((D_FF, D_A), full),
            pl.BlockSpec((1, D_A), full),
            pl.BlockSpec((D_A, N_ASPECTS * D_K), full),
            pl.BlockSpec((D_POOL, N_ASPECTS * D_K), full),
        ],
        out_specs=[
            pl.BlockSpec((TT, D_A), tile),
            pl.BlockSpec((1, D_POOL), full),
        ],
        out_shape=[jax.ShapeDtypeStruct((T, D_A), _F32),
                   jax.ShapeDtypeStruct((1, D_POOL), _F32)],
        scratch_shapes=[pltpu.VMEM((1, D_A), _F32)],
    )(x, attn, wo, s2, b2, w1, bb1, w2, bb2, wq2, wk2)


def _woffn_b_call(x, attn, wo, s2, b2, w1, bb1, w2, bb2):
    def body(x_ref, a_ref, wo_ref, s_ref, b_ref, w1_ref, b1_ref,
             w2_ref, b2_ref, y_ref):
        x1 = x_ref[...] + jnp.dot(a_ref[...], wo_ref[...],
                                  preferred_element_type=_F32)
        h2 = _ln_in(x1, s_ref[...], b_ref[...])
        ff = jax.nn.gelu(jnp.dot(h2, w1_ref[...],
                                 preferred_element_type=_F32) + b1_ref[...])
        y_ref[...] = x1 + jnp.dot(ff, w2_ref[...],
                                  preferred_element_type=_F32) + b2_ref[...]

    tile = lambda i: (i, 0)
    full = lambda i: (0, 0)
    return pl.pallas_call(
        body,
        grid=(NT,),
        in_specs=[
            pl.BlockSpec((TT, D_A), tile),
            pl.BlockSpec((TT, D_A), tile),
            pl.BlockSpec((D_A, D_A), full),
            pl.BlockSpec((1, D_A), full),
            pl.BlockSpec((1, D_A), full),
            pl.BlockSpec((D_A, D_FF), full),
            pl.BlockSpec((1, D_FF), full),
            pl.BlockSpec((D_FF, D_A), full),
            pl.BlockSpec((1, D_A), full),
        ],
        out_specs=pl.BlockSpec((TT, D_A), tile),
        out_shape=jax.ShapeDtypeStruct((T, D_A), _F32),
    )(x, attn, wo, s2, b2, w1, bb1, w2, bb2)


def _score_topk_call(pool, m, lam, warm):
    def body(p_ref, m_ref, lam_ref, warm_ref, a_ref, i_ref, c_scr):
        i = pl.program_id(0)
        c_scr[pl.ds(i, 1), :] = lax.dot_general(
            m_ref[...], p_ref[...], (((1,), (1,)), ((), ())),
            preferred_element_type=_F32)

        @pl.when(i == NPT - 1)
        def _():
            c = c_scr[...] * lam_ref[0]
            cmax = jnp.max(c)
            e = jnp.exp(c - cmax)
            soft = e / jnp.sum(e)
            flat = (lax.broadcasted_iota(jnp.int32, (NPT, PT), 0) * PT
                    + lax.broadcasted_iota(jnp.int32, (NPT, PT), 1))
            cur = soft
            vals = []
            for kk in range(TOP_K):
                mx = jnp.max(cur)
                am = jnp.min(jnp.where(cur == mx, flat, jnp.int32(N_POOL)))
                vals.append(mx)
                i_ref[kk] = am
                cur = jnp.where(flat == am, _F32(-1.0), cur)
            vsum = vals[0]
            for kk in range(1, TOP_K):
                vsum = vsum + vals[kk]
            warmb = warm_ref[0] != 0
            for kk in range(TOP_K):
                a_ref[kk] = jnp.where(warmb, vals[kk],
                                      vals[kk] / (vsum + 1e-9))

    return pl.pallas_call(
        body,
        grid=(NPT,),
        in_specs=[
            pl.BlockSpec((PT, D_POOL), lambda i: (i, 0)),
            pl.BlockSpec((1, D_POOL), lambda i: (0, 0)),
            pl.BlockSpec(memory_space=pltpu.SMEM),
            pl.BlockSpec(memory_space=pltpu.SMEM),
        ],
        out_specs=[
            pl.BlockSpec(memory_space=pltpu.SMEM),
            pl.BlockSpec(memory_space=pltpu.SMEM),
        ],
        out_shape=[
            jax.ShapeDtypeStruct((TOP_K,), _F32),
            jax.ShapeDtypeStruct((TOP_K,), jnp.int32),
        ],
        scratch_shapes=[pltpu.VMEM((NPT, PT), _F32)],
    )(pool, m, lam, warm)


def _lmhead_call(x, w):
    def body(x_ref, w_ref, o_ref):
        o_ref[...] = jnp.dot(x_ref[...], w_ref[...],
                             preferred_element_type=_F32)

    return pl.pallas_call(
        body,
        grid=(NVT,),
        in_specs=[
            pl.BlockSpec((T, D_B), lambda j: (0, 0)),
            pl.BlockSpec((D_B, VT), lambda j: (0, j)),
        ],
        out_specs=pl.BlockSpec((T, VT), lambda j: (0, j)),
        out_shape=jax.ShapeDtypeStruct((T, VOCAB), _F32),
    )(x, w)


def kernel(input_ids, lambda_val, is_warmup, embed_table, a_ln1_s, a_ln1_b,
           a_wqkv, a_wo, a_ln2_s, a_ln2_b, a_w1, a_b1, a_w2, a_b2,
           pool_vectors, w_key, w_query, w_base, b_base, gamma, asm_ln_s,
           asm_ln_b, b_ln1_s, b_ln1_b, b_wqkv, b_wo, b_ln2_s, b_ln2_b,
           b_w1, b_b1, b_w2, b_b2, lm_head_w):
    row2 = lambda a: jnp.asarray(a, _F32).reshape(1, -1)

    ids = input_ids.reshape(T).astype(jnp.int32)
    g = _embed_gather(embed_table, ids)
    pos = jnp.asarray(_POS_ENC)

    attn, x = _qkvattn_a_call(g, pos, row2(a_ln1_s), row2(a_ln1_b), a_wqkv)
    wq2 = w_query.transpose(1, 0, 2).reshape(D_A, N_ASPECTS * D_K)
    wk2 = w_key.transpose(1, 0, 2).reshape(D_POOL, N_ASPECTS * D_K)
    h_a, m = _woffn_a_call(x, attn, a_wo, row2(a_ln2_s), row2(a_ln2_b),
                           a_w1, row2(a_b1), a_w2, row2(a_b2), wq2, wk2)

    lam = jnp.asarray(lambda_val, _F32).reshape(1)
    warm = jnp.asarray(is_warmup, jnp.int32).reshape(1)
    alphas, indices = _score_topk_call(pool_vectors, m, lam, warm)

    gathered = _pool_gather(pool_vectors, indices)
    au = gathered[:, :D_B * R].reshape(TOP_K, D_B, R).transpose(1, 0, 2)
    au = au.reshape(D_B, TOP_K * R)
    bv = gathered[:, D_B * R:].reshape(TOP_K * R, D_A)
    alpha16 = jnp.repeat(alphas, R).reshape(1, TOP_K * R)
    attn2, h_mid = _qkvattn_b_call(h_a, au, bv, alpha16, w_base,
                                   gamma.reshape(1), row2(b_base),
                                   row2(asm_ln_s), row2(asm_ln_b),
                                   row2(b_ln1_s), row2(b_ln1_b), b_wqkv)
    h_out = _woffn_b_call(h_mid, attn2, b_wo, row2(b_ln2_s), row2(b_ln2_b),
                          b_w1, row2(b_b1), b_w2, row2(b_b2))

    logits = _lmhead_call(h_out, lm_head_w)
    return logits.reshape(1, T, VOCAB)
